# Initial kernel scaffold; baseline (speedup 1.0000x reference)
import jax
import jax.numpy as jnp
from jax.experimental import pallas as pl


def kernel(x, edge_index, edge_attr, sampled_edge_index, eps, params):
    raise NotImplementedError("write your pallas kernel here")


# speedup vs baseline: 4.9267x; 4.9267x over previous
import functools

import jax
import jax.numpy as jnp
from jax import lax
from jax.experimental import pallas as pl
from jax.experimental.pallas import tpu as pltpu
from jax.experimental.pallas import tpu_sc as plsc

N = 10000
E = 320000
ES = 160000
D_IN = 128
D_H = 128
D_OUT = 64
N_ATOM = 28

NC = 2
NS = 16
NW = NC * NS
CH = 128
K_E = 80
E_PAD = NW * K_E * CH
RPT_E = K_E * CH
N_ACC = 10240
NZR = N_ACC // NS
HALF = ES + 3840
K_S = 80
_MESH = plsc.VectorSubcoreMesh(core_axis_name="c", subcore_axis_name="s")


def _seg_gather_call(src2d, dst2d, table, zeros):
    d = table.shape[1]

    @functools.partial(
        pl.kernel,
        out_type=jax.ShapeDtypeStruct((NC, N_ACC, d), jnp.float32),
        mesh=_MESH,
        scratch_types=[
            pltpu.VMEM((K_E, CH), jnp.int32),
            pltpu.VMEM((K_E, CH), jnp.int32),
            pltpu.VMEM((CH, d), jnp.float32),
            pltpu.SemaphoreType.DMA,
            pltpu.VMEM_SHARED((N_ACC, d), jnp.float32),
        ],
    )
    def k(src_h, dst_h, tab_h, zero_h, out_h, sidx, didx, rows, sem, acc):
        c = lax.axis_index("c")
        s = lax.axis_index("s")
        wid = c * NS + s
        pltpu.sync_copy(zero_h.at[pl.ds(s * NZR, NZR)], acc.at[pl.ds(s * NZR, NZR)])
        pltpu.sync_copy(src_h.at[pl.ds(wid * K_E, K_E)], sidx)
        pltpu.sync_copy(dst_h.at[pl.ds(wid * K_E, K_E)], didx)
        plsc.subcore_barrier()

        @pl.loop(0, K_E)
        def _(j):
            pltpu.async_copy(tab_h.at[sidx.at[j]], rows, sem).wait()
            pltpu.sync_copy(rows, acc.at[didx.at[j]], add=True)

        plsc.subcore_barrier()
        pltpu.sync_copy(acc.at[pl.ds(s * NZR, NZR)],
                        out_h.at[c].at[pl.ds(s * NZR, NZR)])

    return k(src2d, dst2d, table, zeros)


def _seg_linear_call(vals, dst2d, zeros):
    @functools.partial(
        pl.kernel,
        out_type=jax.ShapeDtypeStruct((NC, N_ACC, 128), jnp.float32),
        mesh=_MESH,
        scratch_types=[
            pltpu.VMEM((K_E, CH), jnp.int32),
            pltpu.VMEM((CH, 128), jnp.float32),
            pltpu.SemaphoreType.DMA,
            pltpu.VMEM_SHARED((N_ACC, 128), jnp.float32),
        ],
    )
    def k(val_h, dst_h, zero_h, out_h, didx, rows, sem, acc):
        c = lax.axis_index("c")
        s = lax.axis_index("s")
        wid = c * NS + s
        pltpu.sync_copy(zero_h.at[pl.ds(s * NZR, NZR)], acc.at[pl.ds(s * NZR, NZR)])
        pltpu.sync_copy(dst_h.at[pl.ds(wid * K_E, K_E)], didx)
        plsc.subcore_barrier()

        @pl.loop(0, K_E)
        def _(j):
            pltpu.async_copy(val_h.at[pl.ds(wid * RPT_E + j * CH, CH)], rows,
                             sem).wait()
            pltpu.sync_copy(rows, acc.at[didx.at[j]], add=True)

        plsc.subcore_barrier()
        pltpu.sync_copy(acc.at[pl.ds(s * NZR, NZR)],
                        out_h.at[c].at[pl.ds(s * NZR, NZR)])

    return k(vals, dst2d, zeros)


_EABLK = 8192


def _expand_ea_call(ea8):
    def body(a_ref, o_ref):
        o_ref[...] = jnp.concatenate(
            [a_ref[...], jnp.zeros((_EABLK, 120), jnp.float32)], axis=1)

    return pl.pallas_call(
        body,
        grid=(E_PAD // _EABLK,),
        in_specs=[pl.BlockSpec((_EABLK, 8), lambda i: (i, 0))],
        out_specs=pl.BlockSpec((_EABLK, 128), lambda i: (i, 0)),
        out_shape=jax.ShapeDtypeStruct((E_PAD, 128), jnp.float32),
    )(ea8)


def _pair_gather_call(idx2d, zw):
    @functools.partial(
        pl.kernel,
        out_type=(jax.ShapeDtypeStruct((HALF, 128), jnp.float32),
                  jax.ShapeDtypeStruct((HALF, 128), jnp.float32)),
        mesh=_MESH,
        scratch_types=[
            pltpu.VMEM((K_S, CH), jnp.int32),
            pltpu.VMEM((CH, 128), jnp.float32),
            pltpu.SemaphoreType.DMA,
        ],
    )
    def k(idx_h, tab_h, out0_h, out1_h, gidx, rows, sem):
        c = lax.axis_index("c")
        s = lax.axis_index("s")
        wid = c * NS + s
        half = wid // 16
        rbase = (wid % 16) * (K_S * CH)
        pltpu.sync_copy(idx_h.at[pl.ds(wid * K_S, K_S)], gidx)

        @pl.loop(0, K_S)
        def _(j):
            pltpu.async_copy(tab_h.at[gidx.at[j]], rows, sem).wait()

            @pl.when(half == 0)
            def _():
                pltpu.sync_copy(rows, out0_h.at[pl.ds(rbase + j * CH, CH)])

            @pl.when(half == 1)
            def _():
                pltpu.sync_copy(rows, out1_h.at[pl.ds(rbase + j * CH, CH)])

    return k(idx2d, zw)


def _embed_call(x2, embp):
    def body(x_ref, e_ref, o_ref):
        oh = (x_ref[...] == lax.broadcasted_iota(jnp.int32, (N, 32), 1))
        o_ref[...] = jnp.dot(oh.astype(jnp.float32), e_ref[...],
                             preferred_element_type=jnp.float32)

    return pl.pallas_call(
        body, out_shape=jax.ShapeDtypeStruct((N, D_IN), jnp.float32),
    )(x2, embp)


_NBLK = 2000
_NNB = N // _NBLK


def _nrow_spec(d):
    return pl.BlockSpec((_NBLK, d), lambda i: (i, 0))


def _part_spec(d):
    return pl.BlockSpec((2, _NBLK, d), lambda i: (0, i, 0))


def _w_spec(a):
    return pl.BlockSpec(a.shape, lambda i: tuple(0 for _ in a.shape))


def _sage_dense_call(h, P, Pea, Ws, Wn, We8, b):
    dout = Wn.shape[1]

    def body(h_ref, p_ref, pe_ref, ws_ref, wn_ref, we_ref, b_ref, o_ref):
        sh = p_ref[0] + p_ref[1]
        se = (pe_ref[0] + pe_ref[1])[:, 0:8]
        deg = jnp.maximum(se[:, 4:5], 1.0)
        agg = (sh + jnp.dot(se, we_ref[...],
                            preferred_element_type=jnp.float32)) / deg
        o = (jnp.dot(h_ref[...], ws_ref[...], preferred_element_type=jnp.float32)
             + jnp.dot(agg, wn_ref[...], preferred_element_type=jnp.float32)
             + b_ref[...])
        o_ref[...] = jnp.maximum(o, 0.0)

    return pl.pallas_call(
        body,
        grid=(_NNB,),
        in_specs=[_nrow_spec(h.shape[1]), _part_spec(P.shape[2]),
                  _part_spec(128), _w_spec(Ws), _w_spec(Wn), _w_spec(We8),
                  _w_spec(b)],
        out_specs=_nrow_spec(dout),
        out_shape=jax.ShapeDtypeStruct((N, dout), jnp.float32),
    )(h, P, Pea, Ws, Wn, We8, b)


def _head_call(h2, P2, Pea, eps, mWs, mWn, mWe8, mb, lWs, lWn, lWe8, lb,
               aW1, ab1):
    def body(h_ref, p_ref, pe_ref, eps_ref, mws_ref, mwn_ref, mwe_ref, mb_ref,
             lws_ref, lwn_ref, lwe_ref, lb_ref, aw1_ref, ab1_ref,
             mu_ref, lv_ref, z_ref, y_ref, st_ref):
        i = pl.program_id(0)
        h = h_ref[...]
        sh = p_ref[0] + p_ref[1]
        se = (pe_ref[0] + pe_ref[1])[:, 0:8]
        deg = jnp.maximum(se[:, 4:5], 1.0)
        agg_m = (sh + jnp.dot(se, mwe_ref[...],
                              preferred_element_type=jnp.float32)) / deg
        agg_l = (sh + jnp.dot(se, lwe_ref[...],
                              preferred_element_type=jnp.float32)) / deg
        mu = (jnp.dot(h, mws_ref[...], preferred_element_type=jnp.float32)
              + jnp.dot(agg_m, mwn_ref[...], preferred_element_type=jnp.float32)
              + mb_ref[...])
        lv = (jnp.dot(h, lws_ref[...], preferred_element_type=jnp.float32)
              + jnp.dot(agg_l, lwn_ref[...], preferred_element_type=jnp.float32)
              + lb_ref[...])
        z = mu + eps_ref[...] * jnp.exp(0.5 * lv)
        mu_ref[...] = mu
        lv_ref[...] = lv
        z_ref[...] = jnp.concatenate(
            [z, jnp.zeros((_NBLK, 128 - D_OUT), jnp.float32)], axis=1)
        y = jnp.dot(z, aw1_ref[...], preferred_element_type=jnp.float32) + ab1_ref[...]
        y_ref[...] = y

        @pl.when(i == 0)
        def _():
            st_ref[...] = jnp.zeros_like(st_ref)

        st_ref[0:1, :] += jnp.sum(y, axis=0, keepdims=True)
        st_ref[1:2, :] += jnp.sum(y * y, axis=0, keepdims=True)

    return pl.pallas_call(
        body,
        grid=(_NNB,),
        in_specs=[_nrow_spec(D_H), _part_spec(D_H), _part_spec(128),
                  _nrow_spec(D_OUT), _w_spec(mWs), _w_spec(mWn), _w_spec(mWe8),
                  _w_spec(mb), _w_spec(lWs), _w_spec(lWn), _w_spec(lWe8),
                  _w_spec(lb), _w_spec(aW1), _w_spec(ab1)],
        out_specs=(_nrow_spec(D_OUT), _nrow_spec(D_OUT), _nrow_spec(128),
                   _nrow_spec(2 * D_OUT),
                   pl.BlockSpec((8, 2 * D_OUT), lambda i: (0, 0))),
        out_shape=(
            jax.ShapeDtypeStruct((N, D_OUT), jnp.float32),
            jax.ShapeDtypeStruct((N, D_OUT), jnp.float32),
            jax.ShapeDtypeStruct((N, 128), jnp.float32),
            jax.ShapeDtypeStruct((N, 2 * D_OUT), jnp.float32),
            jax.ShapeDtypeStruct((8, 2 * D_OUT), jnp.float32),
        ),
    )(h2, P2, Pea, eps, mWs, mWn, mWe8, mb, lWs, lWn, lWe8, lb, aW1, ab1)


def _atom_apply_call(y, st, ag, abe, aW2, ab2):
    def body(y_ref, st_ref, ag_ref, abe_ref, aw2_ref, ab2_ref, o_ref):
        y = y_ref[...]
        m = st_ref[0:1, :] / float(N)
        v = st_ref[1:2, :] / float(N) - m * m
        yh = jnp.maximum(ag_ref[...] * (y - m) / jnp.sqrt(v + 1e-5)
                         + abe_ref[...], 0.0)
        o_ref[...] = (jnp.dot(yh, aw2_ref[...], preferred_element_type=jnp.float32)
                      + ab2_ref[...])

    return pl.pallas_call(
        body,
        grid=(_NNB,),
        in_specs=[_nrow_spec(2 * D_OUT),
                  pl.BlockSpec((8, 2 * D_OUT), lambda i: (0, 0)),
                  _w_spec(ag), _w_spec(abe), _w_spec(aW2), _w_spec(ab2)],
        out_specs=_nrow_spec(N_ATOM),
        out_shape=jax.ShapeDtypeStruct((N, N_ATOM), jnp.float32),
    )(y, st, ag, abe, aW2, ab2)


_RBLK = 2000
_NEB = ES // _RBLK


def _edge_stats_call(g, W1a, W1b, b1):
    dh = W1a.shape[1]

    def body(g0_ref, g1_ref, wa_ref, wb_ref, b_ref, o_ref):
        i = pl.program_id(0)
        ga = g0_ref[...][:, 0:D_OUT]
        gb = g1_ref[...][:, 0:D_OUT]
        y = (jnp.dot(ga, wa_ref[...], preferred_element_type=jnp.float32)
             + jnp.dot(gb, wb_ref[...], preferred_element_type=jnp.float32)
             + b_ref[...])

        @pl.when(i == 0)
        def _():
            o_ref[...] = jnp.zeros_like(o_ref)

        o_ref[0:1, :] += jnp.sum(y, axis=0, keepdims=True)
        o_ref[1:2, :] += jnp.sum(y * y, axis=0, keepdims=True)

    return pl.pallas_call(
        body,
        grid=(_NEB,),
        in_specs=[
            pl.BlockSpec((_RBLK, 128), lambda i: (i, 0)),
            pl.BlockSpec((_RBLK, 128), lambda i: (i, 0)),
            pl.BlockSpec((D_OUT, dh), lambda i: (0, 0)),
            pl.BlockSpec((D_OUT, dh), lambda i: (0, 0)),
            pl.BlockSpec((1, dh), lambda i: (0, 0)),
        ],
        out_specs=pl.BlockSpec((8, dh), lambda i: (0, 0)),
        out_shape=jax.ShapeDtypeStruct((8, dh), jnp.float32),
    )(g[0], g[1], W1a, W1b, b1)


def _edge_apply_call(g, stats, W1a, W1b, b1, eg, ebe, W2, b2):
    dh = W1a.shape[1]

    def body(g0_ref, g1_ref, st_ref, wa_ref, wb_ref, b_ref, g_ref, be_ref,
             w2_ref, b2_ref, o_ref):
        ga = g0_ref[...][:, 0:D_OUT]
        gb = g1_ref[...][:, 0:D_OUT]
        y = (jnp.dot(ga, wa_ref[...], preferred_element_type=jnp.float32)
             + jnp.dot(gb, wb_ref[...], preferred_element_type=jnp.float32)
             + b_ref[...])
        m = st_ref[0:1, :] / float(ES)
        v = st_ref[1:2, :] / float(ES) - m * m
        yh = jnp.maximum(g_ref[...] * (y - m) / jnp.sqrt(v + 1e-5) + be_ref[...], 0.0)
        o_ref[...] = (jnp.dot(yh, w2_ref[...], preferred_element_type=jnp.float32)
                      + b2_ref[...])

    return pl.pallas_call(
        body,
        grid=(_NEB,),
        in_specs=[
            pl.BlockSpec((_RBLK, 128), lambda i: (i, 0)),
            pl.BlockSpec((_RBLK, 128), lambda i: (i, 0)),
            pl.BlockSpec((8, dh), lambda i: (0, 0)),
            pl.BlockSpec((D_OUT, dh), lambda i: (0, 0)),
            pl.BlockSpec((D_OUT, dh), lambda i: (0, 0)),
            pl.BlockSpec((1, dh), lambda i: (0, 0)),
            pl.BlockSpec((1, dh), lambda i: (0, 0)),
            pl.BlockSpec((1, dh), lambda i: (0, 0)),
            pl.BlockSpec((dh, 4), lambda i: (0, 0)),
            pl.BlockSpec((1, 4), lambda i: (0, 0)),
        ],
        out_specs=pl.BlockSpec((_RBLK, 4), lambda i: (i, 0)),
        out_shape=jax.ShapeDtypeStruct((ES, 4), jnp.float32),
    )(g[0], g[1], stats, W1a, W1b, b1, eg, ebe, W2, b2)


def kernel(x, edge_index, edge_attr, sampled_edge_index, eps, params):
    p = params
    f32 = jnp.float32

    x2 = x.astype(jnp.int32).reshape(N, 1)
    src = edge_index[0].astype(jnp.int32)
    dst = edge_index[1].astype(jnp.int32)

    padn = E_PAD - E
    pi = jnp.arange(padn, dtype=jnp.int32) % 128
    src2d = jnp.concatenate([src, pi]).reshape(NW * K_E, CH)
    dst2d = jnp.concatenate([dst, N + pi]).reshape(NW * K_E, CH)

    ea8 = jnp.concatenate(
        [edge_attr.astype(f32),
         jnp.ones((E, 1), f32),
         jnp.zeros((E, 3), f32)], axis=1)
    ea8 = jnp.concatenate([ea8, jnp.zeros((padn, 8), f32)], axis=0)

    sp = jnp.arange(HALF - ES, dtype=jnp.int32) % 128
    s0 = sampled_edge_index[0].astype(jnp.int32)
    s1 = sampled_edge_index[1].astype(jnp.int32)
    sall2d = jnp.concatenate([s0, sp, s1, sp]).reshape(NW * K_S, CH)

    z128 = jnp.zeros((N_ACC, D_IN), f32)

    embp = jnp.concatenate([p['emb'], jnp.zeros((32 - N_ATOM, D_IN), f32)], axis=0)

    def we8(w):
        return jnp.concatenate([w, jnp.zeros((4, w.shape[1]), f32)], axis=0)

    def row(v):
        return v.reshape(1, -1)

    h0 = _embed_call(x2, embp)
    eaW = _expand_ea_call(ea8)
    Pea = _seg_linear_call(eaW, dst2d, z128)
    P0 = _seg_gather_call(src2d, dst2d, h0, z128)
    h1 = _sage_dense_call(h0, P0, Pea, p['c1_Ws'], p['c1_Wn'], we8(p['c1_We']),
                          row(p['c1_b']))
    P1 = _seg_gather_call(src2d, dst2d, h1, z128)
    h2 = _sage_dense_call(h1, P1, Pea, p['c2_Ws'], p['c2_Wn'], we8(p['c2_We']),
                          row(p['c2_b']))
    P2 = _seg_gather_call(src2d, dst2d, h2, z128)
    mu, logvar, z, y_at, st_at = _head_call(
        h2, P2, Pea, eps,
        p['mu_Ws'], p['mu_Wn'], we8(p['mu_We']), row(p['mu_b']),
        p['lv_Ws'], p['lv_Wn'], we8(p['lv_We']), row(p['lv_b']),
        p['at_W1'], row(p['at_b1']))
    atom = _atom_apply_call(y_at, st_at, row(p['at_g']), row(p['at_be']),
                            p['at_W2'], row(p['at_b2']))

    g = _pair_gather_call(sall2d, z)
    W1a = p['el_W1'][:D_OUT]
    W1b = p['el_W1'][D_OUT:]
    stats = _edge_stats_call(g, W1a, W1b, row(p['el_b1']))
    epred = _edge_apply_call(g, stats, W1a, W1b, row(p['el_b1']),
                             row(p['el_g']), row(p['el_be']),
                             p['el_W2'], row(p['el_b2']))
    return (atom, epred, mu, logvar)


# speedup vs baseline: 6.1062x; 1.2394x over previous
import functools

import jax
import jax.numpy as jnp
from jax import lax
from jax.experimental import pallas as pl
from jax.experimental.pallas import tpu as pltpu
from jax.experimental.pallas import tpu_sc as plsc

N = 10000
E = 320000
ES = 160000
D_IN = 128
D_H = 128
D_OUT = 64
N_ATOM = 28

NC = 2
NS = 16
NW = NC * NS
CH = 128
K_E = 80
E_PAD = NW * K_E * CH
RPT_E = K_E * CH
N_ACC = 10240
NZR = N_ACC // NS
HALF = ES + 3840
K_S = 80
_MESH = plsc.VectorSubcoreMesh(core_axis_name="c", subcore_axis_name="s")


def _seg_gather_call(src2d, dst2d, table, zeros):
    d = table.shape[1]

    @functools.partial(
        pl.kernel,
        out_type=jax.ShapeDtypeStruct((NC, N_ACC, d), jnp.float32),
        mesh=_MESH,
        scratch_types=[
            pltpu.VMEM((K_E, CH), jnp.int32),
            pltpu.VMEM((K_E // 2, CH), jnp.int32),
            pltpu.VMEM((CH, d), jnp.float32),
            pltpu.VMEM((CH, d), jnp.float32),
            pltpu.SemaphoreType.DMA,
            pltpu.SemaphoreType.DMA,
            pltpu.VMEM_SHARED((N_ACC, d), jnp.float32),
        ],
    )
    def k(src_h, dst_h, tab_h, zero_h, out_h, sidx, didx, ra, rb, sa, sb, acc):
        c = lax.axis_index("c")
        s = lax.axis_index("s")
        wid = c * NS + s
        kh = K_E // 2
        pltpu.sync_copy(zero_h.at[pl.ds(s * NZR, NZR)], acc.at[pl.ds(s * NZR, NZR)])
        pltpu.sync_copy(src_h.at[pl.ds(wid * K_E, K_E)], sidx)
        plsc.subcore_barrier()

        for ph in range(2):
            pltpu.sync_copy(dst_h.at[pl.ds(wid * K_E + ph * kh, kh)], didx)
            pltpu.async_copy(tab_h.at[sidx.at[ph * kh]], ra, sa)

            @pl.loop(0, kh // 2)
            def _(t):
                j = ph * kh + 2 * t
                pltpu.async_copy(tab_h.at[sidx.at[j + 1]], rb, sb)
                pltpu.make_async_copy(tab_h.at[sidx.at[j]], ra, sa).wait()
                pltpu.sync_copy(ra, acc.at[didx.at[2 * t]], add=True)

                @pl.when(2 * t + 2 < kh)
                def _():
                    pltpu.async_copy(tab_h.at[sidx.at[j + 2]], ra, sa)

                pltpu.make_async_copy(tab_h.at[sidx.at[j + 1]], rb, sb).wait()
                pltpu.sync_copy(rb, acc.at[didx.at[2 * t + 1]], add=True)

        plsc.subcore_barrier()
        pltpu.sync_copy(acc.at[pl.ds(s * NZR, NZR)],
                        out_h.at[c].at[pl.ds(s * NZR, NZR)])

    return k(src2d, dst2d, table, zeros)


def _seg_linear_call(vals, dst2d, zeros):
    @functools.partial(
        pl.kernel,
        out_type=jax.ShapeDtypeStruct((NC, N_ACC, 128), jnp.float32),
        mesh=_MESH,
        scratch_types=[
            pltpu.VMEM((K_E, CH), jnp.int32),
            pltpu.VMEM((CH, 128), jnp.float32),
            pltpu.VMEM((CH, 128), jnp.float32),
            pltpu.SemaphoreType.DMA,
            pltpu.SemaphoreType.DMA,
            pltpu.VMEM_SHARED((N_ACC, 128), jnp.float32),
        ],
    )
    def k(val_h, dst_h, zero_h, out_h, didx, ra, rb, sa, sb, acc):
        c = lax.axis_index("c")
        s = lax.axis_index("s")
        wid = c * NS + s
        base = wid * RPT_E
        pltpu.sync_copy(zero_h.at[pl.ds(s * NZR, NZR)], acc.at[pl.ds(s * NZR, NZR)])
        pltpu.sync_copy(dst_h.at[pl.ds(wid * K_E, K_E)], didx)
        plsc.subcore_barrier()
        pltpu.async_copy(val_h.at[pl.ds(base, CH)], ra, sa)

        @pl.loop(0, K_E // 2)
        def _(t):
            j = 2 * t
            pltpu.async_copy(val_h.at[pl.ds(base + (j + 1) * CH, CH)], rb, sb)
            pltpu.make_async_copy(val_h.at[pl.ds(base, CH)], ra, sa).wait()
            pltpu.sync_copy(ra, acc.at[didx.at[j]], add=True)

            @pl.when(j + 2 < K_E)
            def _():
                pltpu.async_copy(val_h.at[pl.ds(base + (j + 2) * CH, CH)], ra, sa)

            pltpu.make_async_copy(val_h.at[pl.ds(base, CH)], rb, sb).wait()
            pltpu.sync_copy(rb, acc.at[didx.at[j + 1]], add=True)

        plsc.subcore_barrier()
        pltpu.sync_copy(acc.at[pl.ds(s * NZR, NZR)],
                        out_h.at[c].at[pl.ds(s * NZR, NZR)])

    return k(vals, dst2d, zeros)


_EABLK = 8192


def _expand_ea_call(ea8):
    def body(a_ref, o_ref):
        o_ref[...] = jnp.concatenate(
            [a_ref[...], jnp.zeros((_EABLK, 120), jnp.float32)], axis=1)

    return pl.pallas_call(
        body,
        grid=(E_PAD // _EABLK,),
        in_specs=[pl.BlockSpec((_EABLK, 8), lambda i: (i, 0))],
        out_specs=pl.BlockSpec((_EABLK, 128), lambda i: (i, 0)),
        out_shape=jax.ShapeDtypeStruct((E_PAD, 128), jnp.float32),
    )(ea8)


def _pair_gather_call(idx2d, zw):
    @functools.partial(
        pl.kernel,
        out_type=(jax.ShapeDtypeStruct((HALF, 128), jnp.float32),
                  jax.ShapeDtypeStruct((HALF, 128), jnp.float32)),
        mesh=_MESH,
        scratch_types=[
            pltpu.VMEM((K_S, CH), jnp.int32),
            pltpu.VMEM((CH, 128), jnp.float32),
            pltpu.VMEM((CH, 128), jnp.float32),
            pltpu.SemaphoreType.DMA,
            pltpu.SemaphoreType.DMA,
        ],
    )
    def k(idx_h, tab_h, out0_h, out1_h, gidx, ra, rb, sa, sb):
        c = lax.axis_index("c")
        s = lax.axis_index("s")
        wid = c * NS + s
        half = wid // 16
        rbase = (wid % 16) * (K_S * CH)
        pltpu.sync_copy(idx_h.at[pl.ds(wid * K_S, K_S)], gidx)
        pltpu.async_copy(tab_h.at[gidx.at[0]], ra, sa)

        def wr(buf, j):
            @pl.when(half == 0)
            def _():
                pltpu.sync_copy(buf, out0_h.at[pl.ds(rbase + j * CH, CH)])

            @pl.when(half == 1)
            def _():
                pltpu.sync_copy(buf, out1_h.at[pl.ds(rbase + j * CH, CH)])

        @pl.loop(0, K_S // 2)
        def _(t):
            j = 2 * t
            pltpu.async_copy(tab_h.at[gidx.at[j + 1]], rb, sb)
            pltpu.make_async_copy(tab_h.at[gidx.at[j]], ra, sa).wait()
            wr(ra, j)

            @pl.when(j + 2 < K_S)
            def _():
                pltpu.async_copy(tab_h.at[gidx.at[j + 2]], ra, sa)

            pltpu.make_async_copy(tab_h.at[gidx.at[j + 1]], rb, sb).wait()
            wr(rb, j + 1)

    return k(idx2d, zw)


def _embed_call(x2, embp):
    def body(x_ref, e_ref, o_ref):
        oh = (x_ref[...] == lax.broadcasted_iota(jnp.int32, (N, 32), 1))
        o_ref[...] = jnp.dot(oh.astype(jnp.float32), e_ref[...],
                             preferred_element_type=jnp.float32)

    return pl.pallas_call(
        body, out_shape=jax.ShapeDtypeStruct((N, D_IN), jnp.float32),
    )(x2, embp)


_NBLK = 2000
_NNB = N // _NBLK


def _nrow_spec(d):
    return pl.BlockSpec((_NBLK, d), lambda i: (i, 0))


def _part_spec(d):
    return pl.BlockSpec((2, _NBLK, d), lambda i: (0, i, 0))


def _w_spec(a):
    return pl.BlockSpec(a.shape, lambda i: tuple(0 for _ in a.shape))


def _sage_dense_call(h, P, Pea, Ws, Wn, We8, b):
    dout = Wn.shape[1]

    def body(h_ref, p_ref, pe_ref, ws_ref, wn_ref, we_ref, b_ref, o_ref):
        sh = p_ref[0] + p_ref[1]
        se = (pe_ref[0] + pe_ref[1])[:, 0:8]
        deg = jnp.maximum(se[:, 4:5], 1.0)
        agg = (sh + jnp.dot(se, we_ref[...],
                            preferred_element_type=jnp.float32)) / deg
        o = (jnp.dot(h_ref[...], ws_ref[...], preferred_element_type=jnp.float32)
             + jnp.dot(agg, wn_ref[...], preferred_element_type=jnp.float32)
             + b_ref[...])
        o_ref[...] = jnp.maximum(o, 0.0)

    return pl.pallas_call(
        body,
        grid=(_NNB,),
        in_specs=[_nrow_spec(h.shape[1]), _part_spec(P.shape[2]),
                  _part_spec(128), _w_spec(Ws), _w_spec(Wn), _w_spec(We8),
                  _w_spec(b)],
        out_specs=_nrow_spec(dout),
        out_shape=jax.ShapeDtypeStruct((N, dout), jnp.float32),
    )(h, P, Pea, Ws, Wn, We8, b)


def _head_call(h2, P2, Pea, eps, mWs, mWn, mWe8, mb, lWs, lWn, lWe8, lb,
               aW1, ab1):
    def body(h_ref, p_ref, pe_ref, eps_ref, mws_ref, mwn_ref, mwe_ref, mb_ref,
             lws_ref, lwn_ref, lwe_ref, lb_ref, aw1_ref, ab1_ref,
             mu_ref, lv_ref, z_ref, y_ref, st_ref):
        i = pl.program_id(0)
        h = h_ref[...]
        sh = p_ref[0] + p_ref[1]
        se = (pe_ref[0] + pe_ref[1])[:, 0:8]
        deg = jnp.maximum(se[:, 4:5], 1.0)
        agg_m = (sh + jnp.dot(se, mwe_ref[...],
                              preferred_element_type=jnp.float32)) / deg
        agg_l = (sh + jnp.dot(se, lwe_ref[...],
                              preferred_element_type=jnp.float32)) / deg
        mu = (jnp.dot(h, mws_ref[...], preferred_element_type=jnp.float32)
              + jnp.dot(agg_m, mwn_ref[...], preferred_element_type=jnp.float32)
              + mb_ref[...])
        lv = (jnp.dot(h, lws_ref[...], preferred_element_type=jnp.float32)
              + jnp.dot(agg_l, lwn_ref[...], preferred_element_type=jnp.float32)
              + lb_ref[...])
        z = mu + eps_ref[...] * jnp.exp(0.5 * lv)
        mu_ref[...] = mu
        lv_ref[...] = lv
        z_ref[...] = jnp.concatenate(
            [z, jnp.zeros((_NBLK, 128 - D_OUT), jnp.float32)], axis=1)
        y = jnp.dot(z, aw1_ref[...], preferred_element_type=jnp.float32) + ab1_ref[...]
        y_ref[...] = y

        @pl.when(i == 0)
        def _():
            st_ref[...] = jnp.zeros_like(st_ref)

        st_ref[0:1, :] += jnp.sum(y, axis=0, keepdims=True)
        st_ref[1:2, :] += jnp.sum(y * y, axis=0, keepdims=True)

    return pl.pallas_call(
        body,
        grid=(_NNB,),
        in_specs=[_nrow_spec(D_H), _part_spec(D_H), _part_spec(128),
                  _nrow_spec(D_OUT), _w_spec(mWs), _w_spec(mWn), _w_spec(mWe8),
                  _w_spec(mb), _w_spec(lWs), _w_spec(lWn), _w_spec(lWe8),
                  _w_spec(lb), _w_spec(aW1), _w_spec(ab1)],
        out_specs=(_nrow_spec(D_OUT), _nrow_spec(D_OUT), _nrow_spec(128),
                   _nrow_spec(2 * D_OUT),
                   pl.BlockSpec((8, 2 * D_OUT), lambda i: (0, 0))),
        out_shape=(
            jax.ShapeDtypeStruct((N, D_OUT), jnp.float32),
            jax.ShapeDtypeStruct((N, D_OUT), jnp.float32),
            jax.ShapeDtypeStruct((N, 128), jnp.float32),
            jax.ShapeDtypeStruct((N, 2 * D_OUT), jnp.float32),
            jax.ShapeDtypeStruct((8, 2 * D_OUT), jnp.float32),
        ),
    )(h2, P2, Pea, eps, mWs, mWn, mWe8, mb, lWs, lWn, lWe8, lb, aW1, ab1)


def _atom_apply_call(y, st, ag, abe, aW2, ab2):
    def body(y_ref, st_ref, ag_ref, abe_ref, aw2_ref, ab2_ref, o_ref):
        y = y_ref[...]
        m = st_ref[0:1, :] / float(N)
        v = st_ref[1:2, :] / float(N) - m * m
        yh = jnp.maximum(ag_ref[...] * (y - m) / jnp.sqrt(v + 1e-5)
                         + abe_ref[...], 0.0)
        o_ref[...] = (jnp.dot(yh, aw2_ref[...], preferred_element_type=jnp.float32)
                      + ab2_ref[...])

    return pl.pallas_call(
        body,
        grid=(_NNB,),
        in_specs=[_nrow_spec(2 * D_OUT),
                  pl.BlockSpec((8, 2 * D_OUT), lambda i: (0, 0)),
                  _w_spec(ag), _w_spec(abe), _w_spec(aW2), _w_spec(ab2)],
        out_specs=_nrow_spec(N_ATOM),
        out_shape=jax.ShapeDtypeStruct((N, N_ATOM), jnp.float32),
    )(y, st, ag, abe, aW2, ab2)


_RBLK = 2000
_NEB = ES // _RBLK


def _edge_stats_call(g, W1a, W1b, b1):
    dh = W1a.shape[1]

    def body(g0_ref, g1_ref, wa_ref, wb_ref, b_ref, o_ref):
        i = pl.program_id(0)
        ga = g0_ref[...][:, 0:D_OUT]
        gb = g1_ref[...][:, 0:D_OUT]
        y = (jnp.dot(ga, wa_ref[...], preferred_element_type=jnp.float32)
             + jnp.dot(gb, wb_ref[...], preferred_element_type=jnp.float32)
             + b_ref[...])

        @pl.when(i == 0)
        def _():
            o_ref[...] = jnp.zeros_like(o_ref)

        o_ref[0:1, :] += jnp.sum(y, axis=0, keepdims=True)
        o_ref[1:2, :] += jnp.sum(y * y, axis=0, keepdims=True)

    return pl.pallas_call(
        body,
        grid=(_NEB,),
        in_specs=[
            pl.BlockSpec((_RBLK, 128), lambda i: (i, 0)),
            pl.BlockSpec((_RBLK, 128), lambda i: (i, 0)),
            pl.BlockSpec((D_OUT, dh), lambda i: (0, 0)),
            pl.BlockSpec((D_OUT, dh), lambda i: (0, 0)),
            pl.BlockSpec((1, dh), lambda i: (0, 0)),
        ],
        out_specs=pl.BlockSpec((8, dh), lambda i: (0, 0)),
        out_shape=jax.ShapeDtypeStruct((8, dh), jnp.float32),
    )(g[0], g[1], W1a, W1b, b1)


def _edge_apply_call(g, stats, W1a, W1b, b1, eg, ebe, W2, b2):
    dh = W1a.shape[1]

    def body(g0_ref, g1_ref, st_ref, wa_ref, wb_ref, b_ref, g_ref, be_ref,
             w2_ref, b2_ref, o_ref):
        ga = g0_ref[...][:, 0:D_OUT]
        gb = g1_ref[...][:, 0:D_OUT]
        y = (jnp.dot(ga, wa_ref[...], preferred_element_type=jnp.float32)
             + jnp.dot(gb, wb_ref[...], preferred_element_type=jnp.float32)
             + b_ref[...])
        m = st_ref[0:1, :] / float(ES)
        v = st_ref[1:2, :] / float(ES) - m * m
        yh = jnp.maximum(g_ref[...] * (y - m) / jnp.sqrt(v + 1e-5) + be_ref[...], 0.0)
        o_ref[...] = (jnp.dot(yh, w2_ref[...], preferred_element_type=jnp.float32)
                      + b2_ref[...])

    return pl.pallas_call(
        body,
        grid=(_NEB,),
        in_specs=[
            pl.BlockSpec((_RBLK, 128), lambda i: (i, 0)),
            pl.BlockSpec((_RBLK, 128), lambda i: (i, 0)),
            pl.BlockSpec((8, dh), lambda i: (0, 0)),
            pl.BlockSpec((D_OUT, dh), lambda i: (0, 0)),
            pl.BlockSpec((D_OUT, dh), lambda i: (0, 0)),
            pl.BlockSpec((1, dh), lambda i: (0, 0)),
            pl.BlockSpec((1, dh), lambda i: (0, 0)),
            pl.BlockSpec((1, dh), lambda i: (0, 0)),
            pl.BlockSpec((dh, 4), lambda i: (0, 0)),
            pl.BlockSpec((1, 4), lambda i: (0, 0)),
        ],
        out_specs=pl.BlockSpec((_RBLK, 4), lambda i: (i, 0)),
        out_shape=jax.ShapeDtypeStruct((ES, 4), jnp.float32),
    )(g[0], g[1], stats, W1a, W1b, b1, eg, ebe, W2, b2)


def kernel(x, edge_index, edge_attr, sampled_edge_index, eps, params):
    p = params
    f32 = jnp.float32

    x2 = x.astype(jnp.int32).reshape(N, 1)
    src = edge_index[0].astype(jnp.int32)
    dst = edge_index[1].astype(jnp.int32)

    padn = E_PAD - E
    pi = jnp.arange(padn, dtype=jnp.int32) % 128
    src2d = jnp.concatenate([src, pi]).reshape(NW * K_E, CH)
    dst2d = jnp.concatenate([dst, N + pi]).reshape(NW * K_E, CH)

    ea8 = jnp.concatenate(
        [edge_attr.astype(f32),
         jnp.ones((E, 1), f32),
         jnp.zeros((E, 3), f32)], axis=1)
    ea8 = jnp.concatenate([ea8, jnp.zeros((padn, 8), f32)], axis=0)

    sp = jnp.arange(HALF - ES, dtype=jnp.int32) % 128
    s0 = sampled_edge_index[0].astype(jnp.int32)
    s1 = sampled_edge_index[1].astype(jnp.int32)
    sall2d = jnp.concatenate([s0, sp, s1, sp]).reshape(NW * K_S, CH)

    z128 = jnp.zeros((N_ACC, D_IN), f32)

    embp = jnp.concatenate([p['emb'], jnp.zeros((32 - N_ATOM, D_IN), f32)], axis=0)

    def we8(w):
        return jnp.concatenate([w, jnp.zeros((4, w.shape[1]), f32)], axis=0)

    def row(v):
        return v.reshape(1, -1)

    h0 = _embed_call(x2, embp)
    eaW = _expand_ea_call(ea8)
    Pea = _seg_linear_call(eaW, dst2d, z128)
    P0 = _seg_gather_call(src2d, dst2d, h0, z128)
    h1 = _sage_dense_call(h0, P0, Pea, p['c1_Ws'], p['c1_Wn'], we8(p['c1_We']),
                          row(p['c1_b']))
    P1 = _seg_gather_call(src2d, dst2d, h1, z128)
    h2 = _sage_dense_call(h1, P1, Pea, p['c2_Ws'], p['c2_Wn'], we8(p['c2_We']),
                          row(p['c2_b']))
    P2 = _seg_gather_call(src2d, dst2d, h2, z128)
    mu, logvar, z, y_at, st_at = _head_call(
        h2, P2, Pea, eps,
        p['mu_Ws'], p['mu_Wn'], we8(p['mu_We']), row(p['mu_b']),
        p['lv_Ws'], p['lv_Wn'], we8(p['lv_We']), row(p['lv_b']),
        p['at_W1'], row(p['at_b1']))
    atom = _atom_apply_call(y_at, st_at, row(p['at_g']), row(p['at_be']),
                            p['at_W2'], row(p['at_b2']))

    g = _pair_gather_call(sall2d, z)
    W1a = p['el_W1'][:D_OUT]
    W1b = p['el_W1'][D_OUT:]
    stats = _edge_stats_call(g, W1a, W1b, row(p['el_b1']))
    epred = _edge_apply_call(g, stats, W1a, W1b, row(p['el_b1']),
                             row(p['el_g']), row(p['el_be']),
                             p['el_W2'], row(p['el_b2']))
    return (atom, epred, mu, logvar)


# speedup vs baseline: 6.4418x; 1.0550x over previous
import functools

import jax
import jax.numpy as jnp
from jax import lax
from jax.experimental import pallas as pl
from jax.experimental.pallas import tpu as pltpu
from jax.experimental.pallas import tpu_sc as plsc

N = 10000
E = 320000
ES = 160000
D_IN = 128
D_H = 128
D_OUT = 64
N_ATOM = 28

NC = 2
NS = 16
NW = NC * NS
CH = 128
K_E = 80
E_PAD = NW * K_E * CH
RPT_E = K_E * CH
N_ACC = 10240
NZR = N_ACC // NS
HALF = ES + 3840
K_S = 80
_MESH = plsc.VectorSubcoreMesh(core_axis_name="c", subcore_axis_name="s")


def _seg_gather_call(src2d, dst2d, table, zeros):
    d = table.shape[1]

    @functools.partial(
        pl.kernel,
        out_type=jax.ShapeDtypeStruct((NC, N_ACC, d), jnp.float32),
        mesh=_MESH,
        scratch_types=[
            pltpu.VMEM((K_E, CH), jnp.int32),
            pltpu.VMEM((K_E // 2, CH), jnp.int32),
            pltpu.VMEM((CH, d), jnp.float32),
            pltpu.VMEM((CH, d), jnp.float32),
            pltpu.SemaphoreType.DMA,
            pltpu.SemaphoreType.DMA,
            pltpu.VMEM_SHARED((N_ACC, d), jnp.float32),
        ],
    )
    def k(src_h, dst_h, tab_h, zero_h, out_h, sidx, didx, ra, rb, sa, sb, acc):
        c = lax.axis_index("c")
        s = lax.axis_index("s")
        wid = c * NS + s
        kh = K_E // 2
        pltpu.sync_copy(zero_h.at[pl.ds(s * NZR, NZR)], acc.at[pl.ds(s * NZR, NZR)])
        pltpu.sync_copy(src_h.at[pl.ds(wid * K_E, K_E)], sidx)
        plsc.subcore_barrier()

        for ph in range(2):
            pltpu.sync_copy(dst_h.at[pl.ds(wid * K_E + ph * kh, kh)], didx)
            pltpu.async_copy(tab_h.at[sidx.at[ph * kh]], ra, sa)

            @pl.loop(0, kh // 2)
            def _(t):
                j = ph * kh + 2 * t
                pltpu.async_copy(tab_h.at[sidx.at[j + 1]], rb, sb)
                pltpu.make_async_copy(tab_h.at[sidx.at[j]], ra, sa).wait()
                pltpu.sync_copy(ra, acc.at[didx.at[2 * t]], add=True)

                @pl.when(2 * t + 2 < kh)
                def _():
                    pltpu.async_copy(tab_h.at[sidx.at[j + 2]], ra, sa)

                pltpu.make_async_copy(tab_h.at[sidx.at[j + 1]], rb, sb).wait()
                pltpu.sync_copy(rb, acc.at[didx.at[2 * t + 1]], add=True)

        plsc.subcore_barrier()
        pltpu.sync_copy(acc.at[pl.ds(s * NZR, NZR)],
                        out_h.at[c].at[pl.ds(s * NZR, NZR)])

    return k(src2d, dst2d, table, zeros)


def _seg_linear_call(vals, dst2d, zeros):
    @functools.partial(
        pl.kernel,
        out_type=jax.ShapeDtypeStruct((NC, N_ACC, 128), jnp.float32),
        mesh=_MESH,
        scratch_types=[
            pltpu.VMEM((K_E, CH), jnp.int32),
            pltpu.VMEM((CH, 128), jnp.float32),
            pltpu.VMEM((CH, 128), jnp.float32),
            pltpu.SemaphoreType.DMA,
            pltpu.SemaphoreType.DMA,
            pltpu.VMEM_SHARED((N_ACC, 128), jnp.float32),
        ],
    )
    def k(val_h, dst_h, zero_h, out_h, didx, ra, rb, sa, sb, acc):
        c = lax.axis_index("c")
        s = lax.axis_index("s")
        wid = c * NS + s
        base = wid * RPT_E
        pltpu.sync_copy(zero_h.at[pl.ds(s * NZR, NZR)], acc.at[pl.ds(s * NZR, NZR)])
        pltpu.sync_copy(dst_h.at[pl.ds(wid * K_E, K_E)], didx)
        plsc.subcore_barrier()
        pltpu.async_copy(val_h.at[pl.ds(base, CH)], ra, sa)

        @pl.loop(0, K_E // 2)
        def _(t):
            j = 2 * t
            pltpu.async_copy(val_h.at[pl.ds(base + (j + 1) * CH, CH)], rb, sb)
            pltpu.make_async_copy(val_h.at[pl.ds(base, CH)], ra, sa).wait()
            pltpu.sync_copy(ra, acc.at[didx.at[j]], add=True)

            @pl.when(j + 2 < K_E)
            def _():
                pltpu.async_copy(val_h.at[pl.ds(base + (j + 2) * CH, CH)], ra, sa)

            pltpu.make_async_copy(val_h.at[pl.ds(base, CH)], rb, sb).wait()
            pltpu.sync_copy(rb, acc.at[didx.at[j + 1]], add=True)

        plsc.subcore_barrier()
        pltpu.sync_copy(acc.at[pl.ds(s * NZR, NZR)],
                        out_h.at[c].at[pl.ds(s * NZR, NZR)])

    return k(vals, dst2d, zeros)


_EABLK = 8192


def _expand_ea_call(ea):
    def body(a_ref, o_ref):
        o_ref[...] = jnp.concatenate(
            [a_ref[...],
             jnp.ones((_EABLK, 1), jnp.float32),
             jnp.zeros((_EABLK, 123), jnp.float32)], axis=1)

    return pl.pallas_call(
        body,
        grid=(E_PAD // _EABLK,),
        in_specs=[pl.BlockSpec((_EABLK, 4), lambda i: (i, 0))],
        out_specs=pl.BlockSpec((_EABLK, 128), lambda i: (i, 0)),
        out_shape=jax.ShapeDtypeStruct((E_PAD, 128), jnp.float32),
    )(ea)


def _pair_gather_call(idx2d, zw):
    @functools.partial(
        pl.kernel,
        out_type=(jax.ShapeDtypeStruct((HALF, 128), jnp.float32),
                  jax.ShapeDtypeStruct((HALF, 128), jnp.float32)),
        mesh=_MESH,
        scratch_types=[
            pltpu.VMEM((K_S, CH), jnp.int32),
            pltpu.VMEM((CH, 128), jnp.float32),
            pltpu.VMEM((CH, 128), jnp.float32),
            pltpu.SemaphoreType.DMA,
            pltpu.SemaphoreType.DMA,
        ],
    )
    def k(idx_h, tab_h, out0_h, out1_h, gidx, ra, rb, sa, sb):
        c = lax.axis_index("c")
        s = lax.axis_index("s")
        wid = c * NS + s
        half = wid // 16
        rbase = (wid % 16) * (K_S * CH)
        pltpu.sync_copy(idx_h.at[pl.ds(wid * K_S, K_S)], gidx)
        pltpu.async_copy(tab_h.at[gidx.at[0]], ra, sa)

        def wr(buf, j):
            @pl.when(half == 0)
            def _():
                pltpu.sync_copy(buf, out0_h.at[pl.ds(rbase + j * CH, CH)])

            @pl.when(half == 1)
            def _():
                pltpu.sync_copy(buf, out1_h.at[pl.ds(rbase + j * CH, CH)])

        @pl.loop(0, K_S // 2)
        def _(t):
            j = 2 * t
            pltpu.async_copy(tab_h.at[gidx.at[j + 1]], rb, sb)
            pltpu.make_async_copy(tab_h.at[gidx.at[j]], ra, sa).wait()
            wr(ra, j)

            @pl.when(j + 2 < K_S)
            def _():
                pltpu.async_copy(tab_h.at[gidx.at[j + 2]], ra, sa)

            pltpu.make_async_copy(tab_h.at[gidx.at[j + 1]], rb, sb).wait()
            wr(rb, j + 1)

    return k(idx2d, zw)


def _embed_call(x2, embp):
    def body(x_ref, e_ref, o_ref):
        oh = (x_ref[...] == lax.broadcasted_iota(jnp.int32, (N, 32), 1))
        o_ref[...] = jnp.dot(oh.astype(jnp.float32), e_ref[...],
                             preferred_element_type=jnp.float32)

    return pl.pallas_call(
        body, out_shape=jax.ShapeDtypeStruct((N, D_IN), jnp.float32),
    )(x2, embp)


_NBLK = 2000
_NNB = N // _NBLK


def _nrow_spec(d):
    return pl.BlockSpec((_NBLK, d), lambda i: (i, 0))


def _part_spec(d):
    return pl.BlockSpec((2, _NBLK, d), lambda i: (0, i, 0))


def _w_spec(a):
    return pl.BlockSpec(a.shape, lambda i: tuple(0 for _ in a.shape))


def _sage_dense_call(h, P, Pea, Ws, Wn, We8, b):
    dout = Wn.shape[1]

    def body(h_ref, p_ref, pe_ref, ws_ref, wn_ref, we_ref, b_ref, o_ref):
        sh = p_ref[0] + p_ref[1]
        se = (pe_ref[0] + pe_ref[1])[:, 0:8]
        deg = jnp.maximum(se[:, 4:5], 1.0)
        agg = (sh + jnp.dot(se, we_ref[...],
                            preferred_element_type=jnp.float32)) / deg
        o = (jnp.dot(h_ref[...], ws_ref[...], preferred_element_type=jnp.float32)
             + jnp.dot(agg, wn_ref[...], preferred_element_type=jnp.float32)
             + b_ref[...])
        o_ref[...] = jnp.maximum(o, 0.0)

    return pl.pallas_call(
        body,
        grid=(_NNB,),
        in_specs=[_nrow_spec(h.shape[1]), _part_spec(P.shape[2]),
                  _part_spec(128), _w_spec(Ws), _w_spec(Wn), _w_spec(We8),
                  _w_spec(b)],
        out_specs=_nrow_spec(dout),
        out_shape=jax.ShapeDtypeStruct((N, dout), jnp.float32),
    )(h, P, Pea, Ws, Wn, We8, b)


def _head_call(h2, P2, Pea, eps, mWs, mWn, mWe8, mb, lWs, lWn, lWe8, lb,
               aW1, ab1):
    def body(h_ref, p_ref, pe_ref, eps_ref, mws_ref, mwn_ref, mwe_ref, mb_ref,
             lws_ref, lwn_ref, lwe_ref, lb_ref, aw1_ref, ab1_ref,
             mu_ref, lv_ref, z_ref, y_ref, st_ref):
        i = pl.program_id(0)
        h = h_ref[...]
        sh = p_ref[0] + p_ref[1]
        se = (pe_ref[0] + pe_ref[1])[:, 0:8]
        deg = jnp.maximum(se[:, 4:5], 1.0)
        agg_m = (sh + jnp.dot(se, mwe_ref[...],
                              preferred_element_type=jnp.float32)) / deg
        agg_l = (sh + jnp.dot(se, lwe_ref[...],
                              preferred_element_type=jnp.float32)) / deg
        mu = (jnp.dot(h, mws_ref[...], preferred_element_type=jnp.float32)
              + jnp.dot(agg_m, mwn_ref[...], preferred_element_type=jnp.float32)
              + mb_ref[...])
        lv = (jnp.dot(h, lws_ref[...], preferred_element_type=jnp.float32)
              + jnp.dot(agg_l, lwn_ref[...], preferred_element_type=jnp.float32)
              + lb_ref[...])
        z = mu + eps_ref[...] * jnp.exp(0.5 * lv)
        mu_ref[...] = mu
        lv_ref[...] = lv
        z_ref[...] = jnp.concatenate(
            [z, jnp.zeros((_NBLK, 128 - D_OUT), jnp.float32)], axis=1)
        y = jnp.dot(z, aw1_ref[...], preferred_element_type=jnp.float32) + ab1_ref[...]
        y_ref[...] = y

        @pl.when(i == 0)
        def _():
            st_ref[...] = jnp.zeros_like(st_ref)

        st_ref[0:1, :] += jnp.sum(y, axis=0, keepdims=True)
        st_ref[1:2, :] += jnp.sum(y * y, axis=0, keepdims=True)

    return pl.pallas_call(
        body,
        grid=(_NNB,),
        in_specs=[_nrow_spec(D_H), _part_spec(D_H), _part_spec(128),
                  _nrow_spec(D_OUT), _w_spec(mWs), _w_spec(mWn), _w_spec(mWe8),
                  _w_spec(mb), _w_spec(lWs), _w_spec(lWn), _w_spec(lWe8),
                  _w_spec(lb), _w_spec(aW1), _w_spec(ab1)],
        out_specs=(_nrow_spec(D_OUT), _nrow_spec(D_OUT), _nrow_spec(128),
                   _nrow_spec(2 * D_OUT),
                   pl.BlockSpec((8, 2 * D_OUT), lambda i: (0, 0))),
        out_shape=(
            jax.ShapeDtypeStruct((N, D_OUT), jnp.float32),
            jax.ShapeDtypeStruct((N, D_OUT), jnp.float32),
            jax.ShapeDtypeStruct((N, 128), jnp.float32),
            jax.ShapeDtypeStruct((N, 2 * D_OUT), jnp.float32),
            jax.ShapeDtypeStruct((8, 2 * D_OUT), jnp.float32),
        ),
    )(h2, P2, Pea, eps, mWs, mWn, mWe8, mb, lWs, lWn, lWe8, lb, aW1, ab1)


def _atom_apply_call(y, st, ag, abe, aW2, ab2):
    def body(y_ref, st_ref, ag_ref, abe_ref, aw2_ref, ab2_ref, o_ref):
        y = y_ref[...]
        m = st_ref[0:1, :] / float(N)
        v = st_ref[1:2, :] / float(N) - m * m
        yh = jnp.maximum(ag_ref[...] * (y - m) / jnp.sqrt(v + 1e-5)
                         + abe_ref[...], 0.0)
        o_ref[...] = (jnp.dot(yh, aw2_ref[...], preferred_element_type=jnp.float32)
                      + ab2_ref[...])

    return pl.pallas_call(
        body,
        grid=(_NNB,),
        in_specs=[_nrow_spec(2 * D_OUT),
                  pl.BlockSpec((8, 2 * D_OUT), lambda i: (0, 0)),
                  _w_spec(ag), _w_spec(abe), _w_spec(aW2), _w_spec(ab2)],
        out_specs=_nrow_spec(N_ATOM),
        out_shape=jax.ShapeDtypeStruct((N, N_ATOM), jnp.float32),
    )(y, st, ag, abe, aW2, ab2)


_RBLK = 2000
_NEB = ES // _RBLK


def _edge_stats_call(g, W1a, W1b, b1):
    dh = W1a.shape[1]

    def body(g0_ref, g1_ref, wa_ref, wb_ref, b_ref, o_ref):
        i = pl.program_id(0)
        ga = g0_ref[...][:, 0:D_OUT]
        gb = g1_ref[...][:, 0:D_OUT]
        y = (jnp.dot(ga, wa_ref[...], preferred_element_type=jnp.float32)
             + jnp.dot(gb, wb_ref[...], preferred_element_type=jnp.float32)
             + b_ref[...])

        @pl.when(i == 0)
        def _():
            o_ref[...] = jnp.zeros_like(o_ref)

        o_ref[0:1, :] += jnp.sum(y, axis=0, keepdims=True)
        o_ref[1:2, :] += jnp.sum(y * y, axis=0, keepdims=True)

    return pl.pallas_call(
        body,
        grid=(_NEB,),
        in_specs=[
            pl.BlockSpec((_RBLK, 128), lambda i: (i, 0)),
            pl.BlockSpec((_RBLK, 128), lambda i: (i, 0)),
            pl.BlockSpec((D_OUT, dh), lambda i: (0, 0)),
            pl.BlockSpec((D_OUT, dh), lambda i: (0, 0)),
            pl.BlockSpec((1, dh), lambda i: (0, 0)),
        ],
        out_specs=pl.BlockSpec((8, dh), lambda i: (0, 0)),
        out_shape=jax.ShapeDtypeStruct((8, dh), jnp.float32),
    )(g[0], g[1], W1a, W1b, b1)


def _edge_apply_call(g, stats, W1a, W1b, b1, eg, ebe, W2, b2):
    dh = W1a.shape[1]

    def body(g0_ref, g1_ref, st_ref, wa_ref, wb_ref, b_ref, g_ref, be_ref,
             w2_ref, b2_ref, o_ref):
        ga = g0_ref[...][:, 0:D_OUT]
        gb = g1_ref[...][:, 0:D_OUT]
        y = (jnp.dot(ga, wa_ref[...], preferred_element_type=jnp.float32)
             + jnp.dot(gb, wb_ref[...], preferred_element_type=jnp.float32)
             + b_ref[...])
        m = st_ref[0:1, :] / float(ES)
        v = st_ref[1:2, :] / float(ES) - m * m
        yh = jnp.maximum(g_ref[...] * (y - m) / jnp.sqrt(v + 1e-5) + be_ref[...], 0.0)
        o_ref[...] = (jnp.dot(yh, w2_ref[...], preferred_element_type=jnp.float32)
                      + b2_ref[...])

    return pl.pallas_call(
        body,
        grid=(_NEB,),
        in_specs=[
            pl.BlockSpec((_RBLK, 128), lambda i: (i, 0)),
            pl.BlockSpec((_RBLK, 128), lambda i: (i, 0)),
            pl.BlockSpec((8, dh), lambda i: (0, 0)),
            pl.BlockSpec((D_OUT, dh), lambda i: (0, 0)),
            pl.BlockSpec((D_OUT, dh), lambda i: (0, 0)),
            pl.BlockSpec((1, dh), lambda i: (0, 0)),
            pl.BlockSpec((1, dh), lambda i: (0, 0)),
            pl.BlockSpec((1, dh), lambda i: (0, 0)),
            pl.BlockSpec((dh, 4), lambda i: (0, 0)),
            pl.BlockSpec((1, 4), lambda i: (0, 0)),
        ],
        out_specs=pl.BlockSpec((_RBLK, 4), lambda i: (i, 0)),
        out_shape=jax.ShapeDtypeStruct((ES, 4), jnp.float32),
    )(g[0], g[1], stats, W1a, W1b, b1, eg, ebe, W2, b2)


def kernel(x, edge_index, edge_attr, sampled_edge_index, eps, params):
    p = params
    f32 = jnp.float32

    x2 = x.astype(jnp.int32).reshape(N, 1)
    src = edge_index[0].astype(jnp.int32)
    dst = edge_index[1].astype(jnp.int32)

    padn = E_PAD - E
    pi = jnp.arange(padn, dtype=jnp.int32) % 128
    src2d = jnp.concatenate([src, pi]).reshape(NW * K_E, CH)
    dst2d = jnp.concatenate([dst, N + pi]).reshape(NW * K_E, CH)


    sp = jnp.arange(HALF - ES, dtype=jnp.int32) % 128
    s0 = sampled_edge_index[0].astype(jnp.int32)
    s1 = sampled_edge_index[1].astype(jnp.int32)
    sall2d = jnp.concatenate([s0, sp, s1, sp]).reshape(NW * K_S, CH)

    z128 = jnp.zeros((N_ACC, D_IN), f32)

    embp = jnp.concatenate([p['emb'], jnp.zeros((32 - N_ATOM, D_IN), f32)], axis=0)

    def we8(w):
        return jnp.concatenate([w, jnp.zeros((4, w.shape[1]), f32)], axis=0)

    def row(v):
        return v.reshape(1, -1)

    h0 = _embed_call(x2, embp)
    eaW = _expand_ea_call(edge_attr.astype(f32))
    Pea = _seg_linear_call(eaW, dst2d, z128)
    P0 = _seg_gather_call(src2d, dst2d, h0, z128)
    h1 = _sage_dense_call(h0, P0, Pea, p['c1_Ws'], p['c1_Wn'], we8(p['c1_We']),
                          row(p['c1_b']))
    P1 = _seg_gather_call(src2d, dst2d, h1, z128)
    h2 = _sage_dense_call(h1, P1, Pea, p['c2_Ws'], p['c2_Wn'], we8(p['c2_We']),
                          row(p['c2_b']))
    P2 = _seg_gather_call(src2d, dst2d, h2, z128)
    mu, logvar, z, y_at, st_at = _head_call(
        h2, P2, Pea, eps,
        p['mu_Ws'], p['mu_Wn'], we8(p['mu_We']), row(p['mu_b']),
        p['lv_Ws'], p['lv_Wn'], we8(p['lv_We']), row(p['lv_b']),
        p['at_W1'], row(p['at_b1']))
    atom = _atom_apply_call(y_at, st_at, row(p['at_g']), row(p['at_be']),
                            p['at_W2'], row(p['at_b2']))

    g = _pair_gather_call(sall2d, z)
    W1a = p['el_W1'][:D_OUT]
    W1b = p['el_W1'][D_OUT:]
    stats = _edge_stats_call(g, W1a, W1b, row(p['el_b1']))
    epred = _edge_apply_call(g, stats, W1a, W1b, row(p['el_b1']),
                             row(p['el_g']), row(p['el_be']),
                             p['el_W2'], row(p['el_b2']))
    return (atom, epred, mu, logvar)


# speedup vs baseline: 6.4423x; 1.0001x over previous
import functools

import jax
import jax.numpy as jnp
from jax import lax
from jax.experimental import pallas as pl
from jax.experimental.pallas import tpu as pltpu
from jax.experimental.pallas import tpu_sc as plsc

N = 10000
E = 320000
ES = 160000
D_IN = 128
D_H = 128
D_OUT = 64
N_ATOM = 28

NC = 2
NS = 16
NW = NC * NS
CH = 128
K_E = 80
E_PAD = NW * K_E * CH
RPT_E = K_E * CH
N_ACC = 10240
NZR = N_ACC // NS
HALF = ES + 3840
K_S = 80
_MESH = plsc.VectorSubcoreMesh(core_axis_name="c", subcore_axis_name="s")


def _seg_gather_call(src2d, dst2d, table, zeros):
    d = table.shape[1]

    @functools.partial(
        pl.kernel,
        out_type=jax.ShapeDtypeStruct((NC, N_ACC, d), jnp.float32),
        mesh=_MESH,
        scratch_types=[
            pltpu.VMEM((K_E, CH), jnp.int32),
            pltpu.VMEM((K_E // 2, CH), jnp.int32),
            pltpu.VMEM((CH, d), jnp.float32),
            pltpu.VMEM((CH, d), jnp.float32),
            pltpu.SemaphoreType.DMA,
            pltpu.SemaphoreType.DMA,
            pltpu.VMEM_SHARED((N_ACC, d), jnp.float32),
        ],
    )
    def k(src_h, dst_h, tab_h, zero_h, out_h, sidx, didx, ra, rb, sa, sb, acc):
        c = lax.axis_index("c")
        s = lax.axis_index("s")
        wid = c * NS + s
        kh = K_E // 2
        pltpu.sync_copy(zero_h.at[pl.ds(s * NZR, NZR)], acc.at[pl.ds(s * NZR, NZR)])
        pltpu.sync_copy(src_h.at[pl.ds(wid * K_E, K_E)], sidx)
        plsc.subcore_barrier()

        for ph in range(2):
            pltpu.sync_copy(dst_h.at[pl.ds(wid * K_E + ph * kh, kh)], didx)
            pltpu.async_copy(tab_h.at[sidx.at[ph * kh]], ra, sa)

            @pl.loop(0, kh // 2)
            def _(t):
                j = ph * kh + 2 * t
                pltpu.async_copy(tab_h.at[sidx.at[j + 1]], rb, sb)
                pltpu.make_async_copy(tab_h.at[sidx.at[j]], ra, sa).wait()
                pltpu.sync_copy(ra, acc.at[didx.at[2 * t]], add=True)

                @pl.when(2 * t + 2 < kh)
                def _():
                    pltpu.async_copy(tab_h.at[sidx.at[j + 2]], ra, sa)

                pltpu.make_async_copy(tab_h.at[sidx.at[j + 1]], rb, sb).wait()
                pltpu.sync_copy(rb, acc.at[didx.at[2 * t + 1]], add=True)

        plsc.subcore_barrier()
        pltpu.sync_copy(acc.at[pl.ds(s * NZR, NZR)],
                        out_h.at[c].at[pl.ds(s * NZR, NZR)])

    return k(src2d, dst2d, table, zeros)


def _seg_linear_call(vals, dst2d, zeros):
    @functools.partial(
        pl.kernel,
        out_type=jax.ShapeDtypeStruct((NC, N_ACC, 128), jnp.float32),
        mesh=_MESH,
        scratch_types=[
            pltpu.VMEM((K_E, CH), jnp.int32),
            pltpu.VMEM((CH, 128), jnp.float32),
            pltpu.VMEM((CH, 128), jnp.float32),
            pltpu.SemaphoreType.DMA,
            pltpu.SemaphoreType.DMA,
            pltpu.VMEM_SHARED((N_ACC, 128), jnp.float32),
        ],
    )
    def k(val_h, dst_h, zero_h, out_h, didx, ra, rb, sa, sb, acc):
        c = lax.axis_index("c")
        s = lax.axis_index("s")
        wid = c * NS + s
        base = wid * RPT_E
        pltpu.sync_copy(zero_h.at[pl.ds(s * NZR, NZR)], acc.at[pl.ds(s * NZR, NZR)])
        pltpu.sync_copy(dst_h.at[pl.ds(wid * K_E, K_E)], didx)
        plsc.subcore_barrier()
        pltpu.async_copy(val_h.at[pl.ds(base, CH)], ra, sa)

        @pl.loop(0, K_E // 2)
        def _(t):
            j = 2 * t
            pltpu.async_copy(val_h.at[pl.ds(base + (j + 1) * CH, CH)], rb, sb)
            pltpu.make_async_copy(val_h.at[pl.ds(base, CH)], ra, sa).wait()
            pltpu.sync_copy(ra, acc.at[didx.at[j]], add=True)

            @pl.when(j + 2 < K_E)
            def _():
                pltpu.async_copy(val_h.at[pl.ds(base + (j + 2) * CH, CH)], ra, sa)

            pltpu.make_async_copy(val_h.at[pl.ds(base, CH)], rb, sb).wait()
            pltpu.sync_copy(rb, acc.at[didx.at[j + 1]], add=True)

        plsc.subcore_barrier()
        pltpu.sync_copy(acc.at[pl.ds(s * NZR, NZR)],
                        out_h.at[c].at[pl.ds(s * NZR, NZR)])

    return k(vals, dst2d, zeros)


_EABLK = 8192


def _expand_ea_call(ea):
    def body(a_ref, o_ref):
        o_ref[...] = jnp.concatenate(
            [a_ref[...],
             jnp.ones((_EABLK, 1), jnp.float32),
             jnp.zeros((_EABLK, 123), jnp.float32)], axis=1)

    return pl.pallas_call(
        body,
        grid=(E_PAD // _EABLK,),
        in_specs=[pl.BlockSpec((_EABLK, 4), lambda i: (i, 0))],
        out_specs=pl.BlockSpec((_EABLK, 128), lambda i: (i, 0)),
        out_shape=jax.ShapeDtypeStruct((E_PAD, 128), jnp.float32),
    )(ea)


def _pair_gather_call(idx2d, zw):
    @functools.partial(
        pl.kernel,
        out_type=(jax.ShapeDtypeStruct((HALF, 128), jnp.float32),
                  jax.ShapeDtypeStruct((HALF, 128), jnp.float32)),
        mesh=_MESH,
        scratch_types=[
            pltpu.VMEM((K_S, CH), jnp.int32),
            pltpu.VMEM((CH, 128), jnp.float32),
            pltpu.VMEM((CH, 128), jnp.float32),
            pltpu.SemaphoreType.DMA,
            pltpu.SemaphoreType.DMA,
        ],
    )
    def k(idx_h, tab_h, out0_h, out1_h, gidx, ra, rb, sa, sb):
        c = lax.axis_index("c")
        s = lax.axis_index("s")
        wid = c * NS + s
        half = wid // 16
        rbase = (wid % 16) * (K_S * CH)
        pltpu.sync_copy(idx_h.at[pl.ds(wid * K_S, K_S)], gidx)
        pltpu.async_copy(tab_h.at[gidx.at[0]], ra, sa)

        def wr(buf, j):
            @pl.when(half == 0)
            def _():
                pltpu.sync_copy(buf, out0_h.at[pl.ds(rbase + j * CH, CH)])

            @pl.when(half == 1)
            def _():
                pltpu.sync_copy(buf, out1_h.at[pl.ds(rbase + j * CH, CH)])

        @pl.loop(0, K_S // 2)
        def _(t):
            j = 2 * t
            pltpu.async_copy(tab_h.at[gidx.at[j + 1]], rb, sb)
            pltpu.make_async_copy(tab_h.at[gidx.at[j]], ra, sa).wait()
            wr(ra, j)

            @pl.when(j + 2 < K_S)
            def _():
                pltpu.async_copy(tab_h.at[gidx.at[j + 2]], ra, sa)

            pltpu.make_async_copy(tab_h.at[gidx.at[j + 1]], rb, sb).wait()
            wr(rb, j + 1)

    return k(idx2d, zw)


def _embed_call(x2, embp):
    def body(x_ref, e_ref, o_ref):
        oh = (x_ref[...] == lax.broadcasted_iota(jnp.int32, (N, 32), 1))
        o_ref[...] = jnp.dot(oh.astype(jnp.float32), e_ref[...],
                             preferred_element_type=jnp.float32)

    return pl.pallas_call(
        body, out_shape=jax.ShapeDtypeStruct((N, D_IN), jnp.float32),
    )(x2, embp)


_NBLK = 2000
_NNB = N // _NBLK


def _nrow_spec(d):
    return pl.BlockSpec((_NBLK, d), lambda i: (i, 0))


def _part_spec(d):
    return pl.BlockSpec((2, _NBLK, d), lambda i: (0, i, 0))


def _w_spec(a):
    return pl.BlockSpec(a.shape, lambda i: tuple(0 for _ in a.shape))


def _sage_dense_call(h, P, Pea, Ws, Wn, We8, b):
    dout = Wn.shape[1]

    def body(h_ref, p_ref, pe_ref, ws_ref, wn_ref, we_ref, b_ref, o_ref):
        sh = p_ref[0] + p_ref[1]
        se = (pe_ref[0] + pe_ref[1])[:, 0:8]
        deg = jnp.maximum(se[:, 4:5], 1.0)
        agg = (sh + jnp.dot(se, we_ref[...],
                            preferred_element_type=jnp.float32)) / deg
        o = (jnp.dot(h_ref[...], ws_ref[...], preferred_element_type=jnp.float32)
             + jnp.dot(agg, wn_ref[...], preferred_element_type=jnp.float32)
             + b_ref[...])
        o_ref[...] = jnp.maximum(o, 0.0)

    return pl.pallas_call(
        body,
        grid=(_NNB,),
        in_specs=[_nrow_spec(h.shape[1]), _part_spec(P.shape[2]),
                  _part_spec(128), _w_spec(Ws), _w_spec(Wn), _w_spec(We8),
                  _w_spec(b)],
        out_specs=_nrow_spec(dout),
        out_shape=jax.ShapeDtypeStruct((N, dout), jnp.float32),
    )(h, P, Pea, Ws, Wn, We8, b)


def _head_call(h2, P2, Pea, eps, mWs, mWn, mWe8, mb, lWs, lWn, lWe8, lb,
               aW1, ab1):
    def body(h_ref, p_ref, pe_ref, eps_ref, mws_ref, mwn_ref, mwe_ref, mb_ref,
             lws_ref, lwn_ref, lwe_ref, lb_ref, aw1_ref, ab1_ref,
             mu_ref, lv_ref, z_ref, y_ref, st_ref):
        i = pl.program_id(0)
        h = h_ref[...]
        sh = p_ref[0] + p_ref[1]
        se = (pe_ref[0] + pe_ref[1])[:, 0:8]
        deg = jnp.maximum(se[:, 4:5], 1.0)
        agg_m = (sh + jnp.dot(se, mwe_ref[...],
                              preferred_element_type=jnp.float32)) / deg
        agg_l = (sh + jnp.dot(se, lwe_ref[...],
                              preferred_element_type=jnp.float32)) / deg
        mu = (jnp.dot(h, mws_ref[...], preferred_element_type=jnp.float32)
              + jnp.dot(agg_m, mwn_ref[...], preferred_element_type=jnp.float32)
              + mb_ref[...])
        lv = (jnp.dot(h, lws_ref[...], preferred_element_type=jnp.float32)
              + jnp.dot(agg_l, lwn_ref[...], preferred_element_type=jnp.float32)
              + lb_ref[...])
        z = mu + eps_ref[...] * jnp.exp(0.5 * lv)
        mu_ref[...] = mu
        lv_ref[...] = lv
        z_ref[...] = jnp.concatenate(
            [z, jnp.zeros((_NBLK, 128 - D_OUT), jnp.float32)],
            axis=1)
        y = jnp.dot(z, aw1_ref[...], preferred_element_type=jnp.float32) + ab1_ref[...]
        y_ref[...] = y

        @pl.when(i == 0)
        def _():
            st_ref[...] = jnp.zeros_like(st_ref)

        st_ref[0:1, :] += jnp.sum(y, axis=0, keepdims=True)
        st_ref[1:2, :] += jnp.sum(y * y, axis=0, keepdims=True)

    return pl.pallas_call(
        body,
        grid=(_NNB,),
        in_specs=[_nrow_spec(D_H), _part_spec(D_H), _part_spec(128),
                  _nrow_spec(D_OUT), _w_spec(mWs), _w_spec(mWn), _w_spec(mWe8),
                  _w_spec(mb), _w_spec(lWs), _w_spec(lWn), _w_spec(lWe8),
                  _w_spec(lb), _w_spec(aW1), _w_spec(ab1)],
        out_specs=(_nrow_spec(D_OUT), _nrow_spec(D_OUT), _nrow_spec(128),
                   _nrow_spec(2 * D_OUT),
                   pl.BlockSpec((8, 2 * D_OUT), lambda i: (0, 0))),
        out_shape=(
            jax.ShapeDtypeStruct((N, D_OUT), jnp.float32),
            jax.ShapeDtypeStruct((N, D_OUT), jnp.float32),
            jax.ShapeDtypeStruct((N, 128), jnp.float32),
            jax.ShapeDtypeStruct((N, 2 * D_OUT), jnp.float32),
            jax.ShapeDtypeStruct((8, 2 * D_OUT), jnp.float32),
        ),
    )(h2, P2, Pea, eps, mWs, mWn, mWe8, mb, lWs, lWn, lWe8, lb, aW1, ab1)


def _atom_apply_call(y, st, ag, abe, aW2, ab2):
    def body(y_ref, st_ref, ag_ref, abe_ref, aw2_ref, ab2_ref, o_ref):
        y = y_ref[...]
        m = st_ref[0:1, :] / float(N)
        v = st_ref[1:2, :] / float(N) - m * m
        yh = jnp.maximum(ag_ref[...] * (y - m) / jnp.sqrt(v + 1e-5)
                         + abe_ref[...], 0.0)
        o_ref[...] = (jnp.dot(yh, aw2_ref[...], preferred_element_type=jnp.float32)
                      + ab2_ref[...])

    return pl.pallas_call(
        body,
        grid=(_NNB,),
        in_specs=[_nrow_spec(2 * D_OUT),
                  pl.BlockSpec((8, 2 * D_OUT), lambda i: (0, 0)),
                  _w_spec(ag), _w_spec(abe), _w_spec(aW2), _w_spec(ab2)],
        out_specs=_nrow_spec(N_ATOM),
        out_shape=jax.ShapeDtypeStruct((N, N_ATOM), jnp.float32),
    )(y, st, ag, abe, aW2, ab2)


_RBLK = 2000
_NEB = ES // _RBLK


def _edge_stats_call(g, W1a, W1b, b1):
    dh = W1a.shape[1]

    def body(g0_ref, g1_ref, wa_ref, wb_ref, b_ref, o_ref):
        i = pl.program_id(0)
        ga = g0_ref[...][:, 0:D_OUT]
        gb = g1_ref[...][:, 0:D_OUT]
        y = (jnp.dot(ga, wa_ref[...], preferred_element_type=jnp.float32)
             + jnp.dot(gb, wb_ref[...], preferred_element_type=jnp.float32)
             + b_ref[...])

        @pl.when(i == 0)
        def _():
            o_ref[...] = jnp.zeros_like(o_ref)

        o_ref[0:1, :] += jnp.sum(y, axis=0, keepdims=True)
        o_ref[1:2, :] += jnp.sum(y * y, axis=0, keepdims=True)

    return pl.pallas_call(
        body,
        grid=(_NEB,),
        in_specs=[
            pl.BlockSpec((_RBLK, 128), lambda i: (i, 0)),
            pl.BlockSpec((_RBLK, 128), lambda i: (i, 0)),
            pl.BlockSpec((D_OUT, dh), lambda i: (0, 0)),
            pl.BlockSpec((D_OUT, dh), lambda i: (0, 0)),
            pl.BlockSpec((1, dh), lambda i: (0, 0)),
        ],
        out_specs=pl.BlockSpec((8, dh), lambda i: (0, 0)),
        out_shape=jax.ShapeDtypeStruct((8, dh), jnp.float32),
    )(g[0], g[1], W1a, W1b, b1)


def _edge_apply_call(g, stats, W1a, W1b, b1, eg, ebe, W2, b2):
    dh = W1a.shape[1]

    def body(g0_ref, g1_ref, st_ref, wa_ref, wb_ref, b_ref, g_ref, be_ref,
             w2_ref, b2_ref, o_ref):
        ga = g0_ref[...][:, 0:D_OUT]
        gb = g1_ref[...][:, 0:D_OUT]
        y = (jnp.dot(ga, wa_ref[...], preferred_element_type=jnp.float32)
             + jnp.dot(gb, wb_ref[...], preferred_element_type=jnp.float32)
             + b_ref[...])
        m = st_ref[0:1, :] / float(ES)
        v = st_ref[1:2, :] / float(ES) - m * m
        yh = jnp.maximum(g_ref[...] * (y - m) / jnp.sqrt(v + 1e-5) + be_ref[...], 0.0)
        o_ref[...] = (jnp.dot(yh, w2_ref[...], preferred_element_type=jnp.float32)
                      + b2_ref[...])

    return pl.pallas_call(
        body,
        grid=(_NEB,),
        in_specs=[
            pl.BlockSpec((_RBLK, 128), lambda i: (i, 0)),
            pl.BlockSpec((_RBLK, 128), lambda i: (i, 0)),
            pl.BlockSpec((8, dh), lambda i: (0, 0)),
            pl.BlockSpec((D_OUT, dh), lambda i: (0, 0)),
            pl.BlockSpec((D_OUT, dh), lambda i: (0, 0)),
            pl.BlockSpec((1, dh), lambda i: (0, 0)),
            pl.BlockSpec((1, dh), lambda i: (0, 0)),
            pl.BlockSpec((1, dh), lambda i: (0, 0)),
            pl.BlockSpec((dh, 4), lambda i: (0, 0)),
            pl.BlockSpec((1, 4), lambda i: (0, 0)),
        ],
        out_specs=pl.BlockSpec((_RBLK, 4), lambda i: (i, 0)),
        out_shape=jax.ShapeDtypeStruct((ES, 4), jnp.float32),
    )(g[0], g[1], stats, W1a, W1b, b1, eg, ebe, W2, b2)


def kernel(x, edge_index, edge_attr, sampled_edge_index, eps, params):
    p = params
    f32 = jnp.float32

    x2 = x.astype(jnp.int32).reshape(N, 1)
    src = edge_index[0].astype(jnp.int32)
    dst = edge_index[1].astype(jnp.int32)

    padn = E_PAD - E
    pi = jnp.arange(padn, dtype=jnp.int32) % 128
    src2d = jnp.concatenate([src, pi]).reshape(NW * K_E, CH)
    dst2d = jnp.concatenate([dst, N + pi]).reshape(NW * K_E, CH)


    sp = jnp.arange(HALF - ES, dtype=jnp.int32) % 128
    s0 = sampled_edge_index[0].astype(jnp.int32)
    s1 = sampled_edge_index[1].astype(jnp.int32)
    sall2d = jnp.concatenate([s0, sp, s1, sp]).reshape(NW * K_S, CH)

    z128 = jnp.zeros((N_ACC, D_IN), f32)

    embp = jnp.concatenate([p['emb'], jnp.zeros((32 - N_ATOM, D_IN), f32)], axis=0)

    def we8(w):
        return jnp.concatenate([w, jnp.zeros((4, w.shape[1]), f32)], axis=0)

    def row(v):
        return v.reshape(1, -1)

    h0 = _embed_call(x2, embp)
    Pea = _seg_linear_call(_expand_ea_call(edge_attr.astype(f32)), dst2d, z128)
    P0 = _seg_gather_call(src2d, dst2d, h0, z128)
    h1 = _sage_dense_call(h0, P0, Pea, p['c1_Ws'], p['c1_Wn'], we8(p['c1_We']),
                          row(p['c1_b']))
    P1 = _seg_gather_call(src2d, dst2d, h1, z128)
    h2 = _sage_dense_call(h1, P1, Pea, p['c2_Ws'], p['c2_Wn'], we8(p['c2_We']),
                          row(p['c2_b']))
    P2 = _seg_gather_call(src2d, dst2d, h2, z128)
    mu, logvar, z, y_at, st_at = _head_call(
        h2, P2, Pea, eps,
        p['mu_Ws'], p['mu_Wn'], we8(p['mu_We']), row(p['mu_b']),
        p['lv_Ws'], p['lv_Wn'], we8(p['lv_We']), row(p['lv_b']),
        p['at_W1'], row(p['at_b1']))
    atom = _atom_apply_call(y_at, st_at, row(p['at_g']), row(p['at_be']),
                            p['at_W2'], row(p['at_b2']))

    g = _pair_gather_call(sall2d, z)
    W1a = p['el_W1'][:D_OUT]
    W1b = p['el_W1'][D_OUT:]
    stats = _edge_stats_call(g, W1a, W1b, row(p['el_b1']))
    epred = _edge_apply_call(g, stats, W1a, W1b, row(p['el_b1']),
                             row(p['el_g']), row(p['el_be']),
                             p['el_W2'], row(p['el_b2']))
    return (atom, epred, mu, logvar)


# speedup vs baseline: 6.7743x; 1.0515x over previous
import functools

import jax
import jax.numpy as jnp
from jax import lax
from jax.experimental import pallas as pl
from jax.experimental.pallas import tpu as pltpu
from jax.experimental.pallas import tpu_sc as plsc

N = 10000
E = 320000
ES = 160000
D_IN = 128
D_H = 128
D_OUT = 64
N_ATOM = 28

NC = 2
NS = 16
NW = NC * NS
CH = 128
K_E = 80
E_PAD = NW * K_E * CH
RPT_E = K_E * CH
N_ACC = 10240
NZR = N_ACC // NS
HALF = ES + 3840
K_S = 80
_MESH = plsc.VectorSubcoreMesh(core_axis_name="c", subcore_axis_name="s")


def _seg_gather_call(src2d, dst2d, table, zeros):
    d = table.shape[1]

    @functools.partial(
        pl.kernel,
        out_type=jax.ShapeDtypeStruct((NC, N_ACC, d), jnp.float32),
        mesh=_MESH,
        scratch_types=[
            pltpu.VMEM((K_E, CH), jnp.int32),
            pltpu.VMEM((K_E // 2, CH), jnp.int32),
            pltpu.VMEM((CH, d), jnp.float32),
            pltpu.VMEM((CH, d), jnp.float32),
            pltpu.SemaphoreType.DMA,
            pltpu.SemaphoreType.DMA,
            pltpu.VMEM_SHARED((N_ACC, d), jnp.float32),
        ],
    )
    def k(src_h, dst_h, tab_h, zero_h, out_h, sidx, didx, ra, rb, sa, sb, acc):
        c = lax.axis_index("c")
        s = lax.axis_index("s")
        wid = c * NS + s
        kh = K_E // 2
        pltpu.sync_copy(zero_h.at[pl.ds(s * NZR, NZR)], acc.at[pl.ds(s * NZR, NZR)])
        pltpu.sync_copy(src_h.at[pl.ds(wid * K_E, K_E)], sidx)
        plsc.subcore_barrier()

        for ph in range(2):
            pltpu.sync_copy(dst_h.at[pl.ds(wid * K_E + ph * kh, kh)], didx)
            pltpu.async_copy(tab_h.at[sidx.at[ph * kh]], ra, sa)

            @pl.loop(0, kh // 2)
            def _(t):
                j = ph * kh + 2 * t
                pltpu.async_copy(tab_h.at[sidx.at[j + 1]], rb, sb)
                pltpu.make_async_copy(tab_h.at[sidx.at[j]], ra, sa).wait()
                pltpu.sync_copy(ra, acc.at[didx.at[2 * t]], add=True)

                @pl.when(2 * t + 2 < kh)
                def _():
                    pltpu.async_copy(tab_h.at[sidx.at[j + 2]], ra, sa)

                pltpu.make_async_copy(tab_h.at[sidx.at[j + 1]], rb, sb).wait()
                pltpu.sync_copy(rb, acc.at[didx.at[2 * t + 1]], add=True)

        plsc.subcore_barrier()
        pltpu.sync_copy(acc.at[pl.ds(s * NZR, NZR)],
                        out_h.at[c].at[pl.ds(s * NZR, NZR)])

    return k(src2d, dst2d, table, zeros)


def _seg_linear_call(vals, dst2d, zeros):
    @functools.partial(
        pl.kernel,
        out_type=jax.ShapeDtypeStruct((NC, N_ACC, 128), jnp.float32),
        mesh=_MESH,
        scratch_types=[
            pltpu.VMEM((K_E, CH), jnp.int32),
            pltpu.VMEM((CH, 128), jnp.float32),
            pltpu.VMEM((CH, 128), jnp.float32),
            pltpu.SemaphoreType.DMA,
            pltpu.SemaphoreType.DMA,
            pltpu.VMEM_SHARED((N_ACC, 128), jnp.float32),
        ],
    )
    def k(val_h, dst_h, zero_h, out_h, didx, ra, rb, sa, sb, acc):
        c = lax.axis_index("c")
        s = lax.axis_index("s")
        wid = c * NS + s
        base = wid * RPT_E
        pltpu.sync_copy(zero_h.at[pl.ds(s * NZR, NZR)], acc.at[pl.ds(s * NZR, NZR)])
        pltpu.sync_copy(dst_h.at[pl.ds(wid * K_E, K_E)], didx)
        plsc.subcore_barrier()
        pltpu.async_copy(val_h.at[pl.ds(base, CH)], ra, sa)

        @pl.loop(0, K_E // 2)
        def _(t):
            j = 2 * t
            pltpu.async_copy(val_h.at[pl.ds(base + (j + 1) * CH, CH)], rb, sb)
            pltpu.make_async_copy(val_h.at[pl.ds(base, CH)], ra, sa).wait()
            pltpu.sync_copy(ra, acc.at[didx.at[j]], add=True)

            @pl.when(j + 2 < K_E)
            def _():
                pltpu.async_copy(val_h.at[pl.ds(base + (j + 2) * CH, CH)], ra, sa)

            pltpu.make_async_copy(val_h.at[pl.ds(base, CH)], rb, sb).wait()
            pltpu.sync_copy(rb, acc.at[didx.at[j + 1]], add=True)

        plsc.subcore_barrier()
        pltpu.sync_copy(acc.at[pl.ds(s * NZR, NZR)],
                        out_h.at[c].at[pl.ds(s * NZR, NZR)])

    return k(vals, dst2d, zeros)


_EABLK = 8192


def _expand_ea_call(ea):
    def body(a_ref, o_ref):
        o_ref[...] = jnp.concatenate(
            [a_ref[...],
             jnp.ones((_EABLK, 1), jnp.float32),
             jnp.zeros((_EABLK, 123), jnp.float32)], axis=1)

    return pl.pallas_call(
        body,
        grid=(E_PAD // _EABLK,),
        in_specs=[pl.BlockSpec((_EABLK, 4), lambda i: (i, 0))],
        out_specs=pl.BlockSpec((_EABLK, 128), lambda i: (i, 0)),
        out_shape=jax.ShapeDtypeStruct((E_PAD, 128), jnp.float32),
    )(ea)


def _pair_gather_call(idx2d, zw):
    @functools.partial(
        pl.kernel,
        out_type=(jax.ShapeDtypeStruct((HALF, 128), jnp.float32),
                  jax.ShapeDtypeStruct((HALF, 128), jnp.float32)),
        mesh=_MESH,
        scratch_types=[
            pltpu.VMEM((K_S, CH), jnp.int32),
            pltpu.VMEM((CH, 128), jnp.float32),
            pltpu.VMEM((CH, 128), jnp.float32),
            pltpu.SemaphoreType.DMA,
            pltpu.SemaphoreType.DMA,
        ],
    )
    def k(idx_h, tab_h, out0_h, out1_h, gidx, ra, rb, sa, sb):
        c = lax.axis_index("c")
        s = lax.axis_index("s")
        wid = c * NS + s
        half = wid // 16
        rbase = (wid % 16) * (K_S * CH)
        pltpu.sync_copy(idx_h.at[pl.ds(wid * K_S, K_S)], gidx)
        pltpu.async_copy(tab_h.at[gidx.at[0]], ra, sa)

        def wr(buf, j):
            @pl.when(half == 0)
            def _():
                pltpu.sync_copy(buf, out0_h.at[pl.ds(rbase + j * CH, CH)])

            @pl.when(half == 1)
            def _():
                pltpu.sync_copy(buf, out1_h.at[pl.ds(rbase + j * CH, CH)])

        @pl.loop(0, K_S // 2)
        def _(t):
            j = 2 * t
            pltpu.async_copy(tab_h.at[gidx.at[j + 1]], rb, sb)
            pltpu.make_async_copy(tab_h.at[gidx.at[j]], ra, sa).wait()
            wr(ra, j)

            @pl.when(j + 2 < K_S)
            def _():
                pltpu.async_copy(tab_h.at[gidx.at[j + 2]], ra, sa)

            pltpu.make_async_copy(tab_h.at[gidx.at[j + 1]], rb, sb).wait()
            wr(rb, j + 1)

    return k(idx2d, zw)


def _embed_call(x2, embp):
    def body(x_ref, e_ref, o_ref):
        oh = (x_ref[...] == lax.broadcasted_iota(jnp.int32, (N, 32), 1))
        o_ref[...] = jnp.dot(oh.astype(jnp.float32), e_ref[...],
                             preferred_element_type=jnp.float32)

    return pl.pallas_call(
        body, out_shape=jax.ShapeDtypeStruct((N, D_IN), jnp.float32),
    )(x2, embp)


_NBLK = 2000
_NNB = N // _NBLK


def _nrow_spec(d):
    return pl.BlockSpec((_NBLK, d), lambda i: (i, 0))


def _part_spec(d):
    return pl.BlockSpec((2, _NBLK, d), lambda i: (0, i, 0))


def _w_spec(a):
    return pl.BlockSpec(a.shape, lambda i: tuple(0 for _ in a.shape))


def _sage_dense_call(h, P, Pea, Ws, Wn, We8, b):
    dout = Wn.shape[1]

    def body(h_ref, p_ref, pe_ref, ws_ref, wn_ref, we_ref, b_ref, o_ref):
        sh = p_ref[0] + p_ref[1]
        se = (pe_ref[0] + pe_ref[1])[:, 0:8]
        deg = jnp.maximum(se[:, 4:5], 1.0)
        agg = (sh + jnp.dot(se, we_ref[...],
                            preferred_element_type=jnp.float32)) / deg
        o = (jnp.dot(h_ref[...], ws_ref[...], preferred_element_type=jnp.float32)
             + jnp.dot(agg, wn_ref[...], preferred_element_type=jnp.float32)
             + b_ref[...])
        o_ref[...] = jnp.maximum(o, 0.0)

    return pl.pallas_call(
        body,
        grid=(_NNB,),
        in_specs=[_nrow_spec(h.shape[1]), _part_spec(P.shape[2]),
                  _part_spec(128), _w_spec(Ws), _w_spec(Wn), _w_spec(We8),
                  _w_spec(b)],
        out_specs=_nrow_spec(dout),
        out_shape=jax.ShapeDtypeStruct((N, dout), jnp.float32),
    )(h, P, Pea, Ws, Wn, We8, b)


def _head_call(h2, P2, Pea, eps, mWs, mWn, mWe8, mb, lWs, lWn, lWe8, lb,
               aW1, ab1):
    def body(h_ref, p_ref, pe_ref, eps_ref, mws_ref, mwn_ref, mwe_ref, mb_ref,
             lws_ref, lwn_ref, lwe_ref, lb_ref, aw1_ref, ab1_ref,
             mu_ref, lv_ref, z_ref, y_ref, st_ref):
        i = pl.program_id(0)
        h = h_ref[...]
        sh = p_ref[0] + p_ref[1]
        se = (pe_ref[0] + pe_ref[1])[:, 0:8]
        deg = jnp.maximum(se[:, 4:5], 1.0)
        agg_m = (sh + jnp.dot(se, mwe_ref[...],
                              preferred_element_type=jnp.float32)) / deg
        agg_l = (sh + jnp.dot(se, lwe_ref[...],
                              preferred_element_type=jnp.float32)) / deg
        mu = (jnp.dot(h, mws_ref[...], preferred_element_type=jnp.float32)
              + jnp.dot(agg_m, mwn_ref[...], preferred_element_type=jnp.float32)
              + mb_ref[...])
        lv = (jnp.dot(h, lws_ref[...], preferred_element_type=jnp.float32)
              + jnp.dot(agg_l, lwn_ref[...], preferred_element_type=jnp.float32)
              + lb_ref[...])
        z = mu + eps_ref[...] * jnp.exp(0.5 * lv)
        mu_ref[...] = mu
        lv_ref[...] = lv
        z_ref[...] = jnp.concatenate(
            [z, jnp.zeros((_NBLK, 128 - D_OUT), jnp.float32)],
            axis=1)
        y = jnp.dot(z, aw1_ref[...], preferred_element_type=jnp.float32) + ab1_ref[...]
        y_ref[...] = y

        @pl.when(i == 0)
        def _():
            st_ref[...] = jnp.zeros_like(st_ref)

        st_ref[0:1, :] += jnp.sum(y, axis=0, keepdims=True)
        st_ref[1:2, :] += jnp.sum(y * y, axis=0, keepdims=True)

    return pl.pallas_call(
        body,
        grid=(_NNB,),
        in_specs=[_nrow_spec(D_H), _part_spec(D_H), _part_spec(128),
                  _nrow_spec(D_OUT), _w_spec(mWs), _w_spec(mWn), _w_spec(mWe8),
                  _w_spec(mb), _w_spec(lWs), _w_spec(lWn), _w_spec(lWe8),
                  _w_spec(lb), _w_spec(aW1), _w_spec(ab1)],
        out_specs=(_nrow_spec(D_OUT), _nrow_spec(D_OUT), _nrow_spec(128),
                   _nrow_spec(2 * D_OUT),
                   pl.BlockSpec((8, 2 * D_OUT), lambda i: (0, 0))),
        out_shape=(
            jax.ShapeDtypeStruct((N, D_OUT), jnp.float32),
            jax.ShapeDtypeStruct((N, D_OUT), jnp.float32),
            jax.ShapeDtypeStruct((N, 128), jnp.float32),
            jax.ShapeDtypeStruct((N, 2 * D_OUT), jnp.float32),
            jax.ShapeDtypeStruct((8, 2 * D_OUT), jnp.float32),
        ),
    )(h2, P2, Pea, eps, mWs, mWn, mWe8, mb, lWs, lWn, lWe8, lb, aW1, ab1)


def _atom_apply_call(y, st, ag, abe, aW2, ab2):
    def body(y_ref, st_ref, ag_ref, abe_ref, aw2_ref, ab2_ref, o_ref):
        y = y_ref[...]
        m = st_ref[0:1, :] / float(N)
        v = st_ref[1:2, :] / float(N) - m * m
        yh = jnp.maximum(ag_ref[...] * (y - m) / jnp.sqrt(v + 1e-5)
                         + abe_ref[...], 0.0)
        o_ref[...] = (jnp.dot(yh, aw2_ref[...], preferred_element_type=jnp.float32)
                      + ab2_ref[...])

    return pl.pallas_call(
        body,
        grid=(_NNB,),
        in_specs=[_nrow_spec(2 * D_OUT),
                  pl.BlockSpec((8, 2 * D_OUT), lambda i: (0, 0)),
                  _w_spec(ag), _w_spec(abe), _w_spec(aW2), _w_spec(ab2)],
        out_specs=_nrow_spec(N_ATOM),
        out_shape=jax.ShapeDtypeStruct((N, N_ATOM), jnp.float32),
    )(y, st, ag, abe, aW2, ab2)


_RBLK = 4000
_NEB = ES // _RBLK


def _edge_stats_call(g, W1a, W1b, b1):
    dh = W1a.shape[1]

    def body(g0_ref, g1_ref, wa_ref, wb_ref, b_ref, o_ref):
        i = pl.program_id(0)
        ga = g0_ref[...][:, 0:D_OUT]
        gb = g1_ref[...][:, 0:D_OUT]
        y = (jnp.dot(ga, wa_ref[...], preferred_element_type=jnp.float32)
             + jnp.dot(gb, wb_ref[...], preferred_element_type=jnp.float32)
             + b_ref[...])

        @pl.when(i == 0)
        def _():
            o_ref[...] = jnp.zeros_like(o_ref)

        o_ref[0:1, :] += jnp.sum(y, axis=0, keepdims=True)
        o_ref[1:2, :] += jnp.sum(y * y, axis=0, keepdims=True)

    return pl.pallas_call(
        body,
        grid=(_NEB,),
        in_specs=[
            pl.BlockSpec((_RBLK, 128), lambda i: (i, 0)),
            pl.BlockSpec((_RBLK, 128), lambda i: (i, 0)),
            pl.BlockSpec((D_OUT, dh), lambda i: (0, 0)),
            pl.BlockSpec((D_OUT, dh), lambda i: (0, 0)),
            pl.BlockSpec((1, dh), lambda i: (0, 0)),
        ],
        out_specs=pl.BlockSpec((8, dh), lambda i: (0, 0)),
        out_shape=jax.ShapeDtypeStruct((8, dh), jnp.float32),
    )(g[0], g[1], W1a, W1b, b1)


def _edge_apply_call(g, stats, W1a, W1b, b1, eg, ebe, W2, b2):
    dh = W1a.shape[1]

    def body(g0_ref, g1_ref, st_ref, wa_ref, wb_ref, b_ref, g_ref, be_ref,
             w2_ref, b2_ref, o_ref):
        ga = g0_ref[...][:, 0:D_OUT]
        gb = g1_ref[...][:, 0:D_OUT]
        y = (jnp.dot(ga, wa_ref[...], preferred_element_type=jnp.float32)
             + jnp.dot(gb, wb_ref[...], preferred_element_type=jnp.float32)
             + b_ref[...])
        m = st_ref[0:1, :] / float(ES)
        v = st_ref[1:2, :] / float(ES) - m * m
        yh = jnp.maximum(g_ref[...] * (y - m) / jnp.sqrt(v + 1e-5) + be_ref[...], 0.0)
        o_ref[...] = (jnp.dot(yh, w2_ref[...], preferred_element_type=jnp.float32)
                      + b2_ref[...])

    return pl.pallas_call(
        body,
        grid=(_NEB,),
        in_specs=[
            pl.BlockSpec((_RBLK, 128), lambda i: (i, 0)),
            pl.BlockSpec((_RBLK, 128), lambda i: (i, 0)),
            pl.BlockSpec((8, dh), lambda i: (0, 0)),
            pl.BlockSpec((D_OUT, dh), lambda i: (0, 0)),
            pl.BlockSpec((D_OUT, dh), lambda i: (0, 0)),
            pl.BlockSpec((1, dh), lambda i: (0, 0)),
            pl.BlockSpec((1, dh), lambda i: (0, 0)),
            pl.BlockSpec((1, dh), lambda i: (0, 0)),
            pl.BlockSpec((dh, 4), lambda i: (0, 0)),
            pl.BlockSpec((1, 4), lambda i: (0, 0)),
        ],
        out_specs=pl.BlockSpec((_RBLK, 4), lambda i: (i, 0)),
        out_shape=jax.ShapeDtypeStruct((ES, 4), jnp.float32),
    )(g[0], g[1], stats, W1a, W1b, b1, eg, ebe, W2, b2)


def kernel(x, edge_index, edge_attr, sampled_edge_index, eps, params):
    p = params
    f32 = jnp.float32

    x2 = x.astype(jnp.int32).reshape(N, 1)
    src = edge_index[0].astype(jnp.int32)
    dst = edge_index[1].astype(jnp.int32)

    padn = E_PAD - E
    pi = jnp.arange(padn, dtype=jnp.int32) % 128
    src2d = jnp.concatenate([src, pi]).reshape(NW * K_E, CH)
    dst2d = jnp.concatenate([dst, N + pi]).reshape(NW * K_E, CH)


    sp = jnp.arange(HALF - ES, dtype=jnp.int32) % 128
    s0 = sampled_edge_index[0].astype(jnp.int32)
    s1 = sampled_edge_index[1].astype(jnp.int32)
    sall2d = jnp.concatenate([s0, sp, s1, sp]).reshape(NW * K_S, CH)

    z128 = jnp.zeros((N_ACC, D_IN), f32)

    embp = jnp.concatenate([p['emb'], jnp.zeros((32 - N_ATOM, D_IN), f32)], axis=0)

    def we8(w):
        return jnp.concatenate([w, jnp.zeros((4, w.shape[1]), f32)], axis=0)

    def row(v):
        return v.reshape(1, -1)

    h0 = _embed_call(x2, embp)
    Pea = _seg_linear_call(_expand_ea_call(edge_attr.astype(f32)), dst2d, z128)
    P0 = _seg_gather_call(src2d, dst2d, h0, z128)
    h1 = _sage_dense_call(h0, P0, Pea, p['c1_Ws'], p['c1_Wn'], we8(p['c1_We']),
                          row(p['c1_b']))
    P1 = _seg_gather_call(src2d, dst2d, h1, z128)
    h2 = _sage_dense_call(h1, P1, Pea, p['c2_Ws'], p['c2_Wn'], we8(p['c2_We']),
                          row(p['c2_b']))
    P2 = _seg_gather_call(src2d, dst2d, h2, z128)
    mu, logvar, z, y_at, st_at = _head_call(
        h2, P2, Pea, eps,
        p['mu_Ws'], p['mu_Wn'], we8(p['mu_We']), row(p['mu_b']),
        p['lv_Ws'], p['lv_Wn'], we8(p['lv_We']), row(p['lv_b']),
        p['at_W1'], row(p['at_b1']))
    atom = _atom_apply_call(y_at, st_at, row(p['at_g']), row(p['at_be']),
                            p['at_W2'], row(p['at_b2']))

    g = _pair_gather_call(sall2d, z)
    W1a = p['el_W1'][:D_OUT]
    W1b = p['el_W1'][D_OUT:]
    stats = _edge_stats_call(g, W1a, W1b, row(p['el_b1']))
    epred = _edge_apply_call(g, stats, W1a, W1b, row(p['el_b1']),
                             row(p['el_g']), row(p['el_be']),
                             p['el_W2'], row(p['el_b2']))
    return (atom, epred, mu, logvar)


# speedup vs baseline: 6.9128x; 1.0204x over previous
import functools

import jax
import jax.numpy as jnp
from jax import lax
from jax.experimental import pallas as pl
from jax.experimental.pallas import tpu as pltpu
from jax.experimental.pallas import tpu_sc as plsc

N = 10000
E = 320000
ES = 160000
D_IN = 128
D_H = 128
D_OUT = 64
N_ATOM = 28

NC = 2
NS = 16
NW = NC * NS
CH = 128
K_E = 80
E_PAD = NW * K_E * CH
RPT_E = K_E * CH
N_ACC = 10240
NZR = N_ACC // NS
HALF = ES + 3840
K_S = 80
_MESH = plsc.VectorSubcoreMesh(core_axis_name="c", subcore_axis_name="s")


def _seg_gather_call(src2d, dst2d, table, zeros):
    d = table.shape[1]

    @functools.partial(
        pl.kernel,
        out_type=jax.ShapeDtypeStruct((NC, N_ACC, d), jnp.float32),
        mesh=_MESH,
        scratch_types=[
            pltpu.VMEM((K_E, CH), jnp.int32),
            pltpu.VMEM((K_E // 2, CH), jnp.int32),
            pltpu.VMEM((CH, d), jnp.float32),
            pltpu.VMEM((CH, d), jnp.float32),
            pltpu.SemaphoreType.DMA,
            pltpu.SemaphoreType.DMA,
            pltpu.VMEM_SHARED((N_ACC, d), jnp.float32),
        ],
    )
    def k(src_h, dst_h, tab_h, zero_h, out_h, sidx, didx, ra, rb, sa, sb, acc):
        c = lax.axis_index("c")
        s = lax.axis_index("s")
        wid = c * NS + s
        kh = K_E // 2
        pltpu.sync_copy(zero_h.at[pl.ds(s * NZR, NZR)], acc.at[pl.ds(s * NZR, NZR)])
        pltpu.sync_copy(src_h.at[pl.ds(wid * K_E, K_E)], sidx)
        plsc.subcore_barrier()

        for ph in range(2):
            pltpu.sync_copy(dst_h.at[pl.ds(wid * K_E + ph * kh, kh)], didx)
            pltpu.async_copy(tab_h.at[sidx.at[ph * kh]], ra, sa)

            @pl.loop(0, kh // 2)
            def _(t):
                j = ph * kh + 2 * t
                pltpu.async_copy(tab_h.at[sidx.at[j + 1]], rb, sb)
                pltpu.make_async_copy(tab_h.at[sidx.at[j]], ra, sa).wait()
                pltpu.sync_copy(ra, acc.at[didx.at[2 * t]], add=True)

                @pl.when(2 * t + 2 < kh)
                def _():
                    pltpu.async_copy(tab_h.at[sidx.at[j + 2]], ra, sa)

                pltpu.make_async_copy(tab_h.at[sidx.at[j + 1]], rb, sb).wait()
                pltpu.sync_copy(rb, acc.at[didx.at[2 * t + 1]], add=True)

        plsc.subcore_barrier()
        pltpu.sync_copy(acc.at[pl.ds(s * NZR, NZR)],
                        out_h.at[c].at[pl.ds(s * NZR, NZR)])

    return k(src2d, dst2d, table, zeros)


def _seg_linear_call(vals, dst2d, zeros):
    @functools.partial(
        pl.kernel,
        out_type=jax.ShapeDtypeStruct((NC, N_ACC, 128), jnp.float32),
        mesh=_MESH,
        scratch_types=[
            pltpu.VMEM((K_E, CH), jnp.int32),
            pltpu.VMEM((CH, 128), jnp.float32),
            pltpu.VMEM((CH, 128), jnp.float32),
            pltpu.SemaphoreType.DMA,
            pltpu.SemaphoreType.DMA,
            pltpu.VMEM_SHARED((N_ACC, 128), jnp.float32),
        ],
    )
    def k(val_h, dst_h, zero_h, out_h, didx, ra, rb, sa, sb, acc):
        c = lax.axis_index("c")
        s = lax.axis_index("s")
        wid = c * NS + s
        base = wid * RPT_E
        pltpu.sync_copy(zero_h.at[pl.ds(s * NZR, NZR)], acc.at[pl.ds(s * NZR, NZR)])
        pltpu.sync_copy(dst_h.at[pl.ds(wid * K_E, K_E)], didx)
        plsc.subcore_barrier()
        pltpu.async_copy(val_h.at[pl.ds(base, CH)], ra, sa)

        @pl.loop(0, K_E // 2)
        def _(t):
            j = 2 * t
            pltpu.async_copy(val_h.at[pl.ds(base + (j + 1) * CH, CH)], rb, sb)
            pltpu.make_async_copy(val_h.at[pl.ds(base, CH)], ra, sa).wait()
            pltpu.sync_copy(ra, acc.at[didx.at[j]], add=True)

            @pl.when(j + 2 < K_E)
            def _():
                pltpu.async_copy(val_h.at[pl.ds(base + (j + 2) * CH, CH)], ra, sa)

            pltpu.make_async_copy(val_h.at[pl.ds(base, CH)], rb, sb).wait()
            pltpu.sync_copy(rb, acc.at[didx.at[j + 1]], add=True)

        plsc.subcore_barrier()
        pltpu.sync_copy(acc.at[pl.ds(s * NZR, NZR)],
                        out_h.at[c].at[pl.ds(s * NZR, NZR)])

    return k(vals, dst2d, zeros)


_EABLK = 8192


def _expand_ea_call(ea):
    def body(a_ref, o_ref):
        o_ref[...] = jnp.concatenate(
            [a_ref[...],
             jnp.ones((_EABLK, 1), jnp.float32),
             jnp.zeros((_EABLK, 123), jnp.float32)], axis=1)

    return pl.pallas_call(
        body,
        grid=(E_PAD // _EABLK,),
        in_specs=[pl.BlockSpec((_EABLK, 4), lambda i: (i, 0))],
        out_specs=pl.BlockSpec((_EABLK, 128), lambda i: (i, 0)),
        out_shape=jax.ShapeDtypeStruct((E_PAD, 128), jnp.float32),
    )(ea)


def _pair_gather_call(idx2d, zw):
    @functools.partial(
        pl.kernel,
        out_type=(jax.ShapeDtypeStruct((HALF, 128), jnp.float32),
                  jax.ShapeDtypeStruct((HALF, 128), jnp.float32)),
        mesh=_MESH,
        scratch_types=[
            pltpu.VMEM((K_S, CH), jnp.int32),
            pltpu.VMEM((CH, 128), jnp.float32),
            pltpu.VMEM((CH, 128), jnp.float32),
            pltpu.SemaphoreType.DMA,
            pltpu.SemaphoreType.DMA,
        ],
    )
    def k(idx_h, tab_h, out0_h, out1_h, gidx, ra, rb, sa, sb):
        c = lax.axis_index("c")
        s = lax.axis_index("s")
        wid = c * NS + s
        half = wid // 16
        rbase = (wid % 16) * (K_S * CH)
        pltpu.sync_copy(idx_h.at[pl.ds(wid * K_S, K_S)], gidx)
        pltpu.async_copy(tab_h.at[gidx.at[0]], ra, sa)

        def wr(buf, j):
            @pl.when(half == 0)
            def _():
                pltpu.sync_copy(buf, out0_h.at[pl.ds(rbase + j * CH, CH)])

            @pl.when(half == 1)
            def _():
                pltpu.sync_copy(buf, out1_h.at[pl.ds(rbase + j * CH, CH)])

        @pl.loop(0, K_S // 2)
        def _(t):
            j = 2 * t
            pltpu.async_copy(tab_h.at[gidx.at[j + 1]], rb, sb)
            pltpu.make_async_copy(tab_h.at[gidx.at[j]], ra, sa).wait()
            wr(ra, j)

            @pl.when(j + 2 < K_S)
            def _():
                pltpu.async_copy(tab_h.at[gidx.at[j + 2]], ra, sa)

            pltpu.make_async_copy(tab_h.at[gidx.at[j + 1]], rb, sb).wait()
            wr(rb, j + 1)

    return k(idx2d, zw)


def _embed_call(x2, embp):
    def body(x_ref, e_ref, o_ref):
        oh = (x_ref[...] == lax.broadcasted_iota(jnp.int32, (N, 32), 1))
        o_ref[...] = jnp.dot(oh.astype(jnp.float32), e_ref[...],
                             preferred_element_type=jnp.float32)

    return pl.pallas_call(
        body, out_shape=jax.ShapeDtypeStruct((N, D_IN), jnp.float32),
    )(x2, embp)


_NBLK = 2000
_NNB = N // _NBLK


def _nrow_spec(d):
    return pl.BlockSpec((_NBLK, d), lambda i: (i, 0))


def _part_spec(d):
    return pl.BlockSpec((2, _NBLK, d), lambda i: (0, i, 0))


def _w_spec(a):
    return pl.BlockSpec(a.shape, lambda i: tuple(0 for _ in a.shape))


def _sage_dense_call(h, P, Pea, Ws, Wn, We8, b):
    dout = Wn.shape[1]

    def body(h_ref, p_ref, pe_ref, ws_ref, wn_ref, we_ref, b_ref, o_ref):
        sh = p_ref[0] + p_ref[1]
        se = (pe_ref[0] + pe_ref[1])[:, 0:8]
        deg = jnp.maximum(se[:, 4:5], 1.0)
        agg = (sh + jnp.dot(se, we_ref[...],
                            preferred_element_type=jnp.float32)) / deg
        o = (jnp.dot(h_ref[...], ws_ref[...], preferred_element_type=jnp.float32)
             + jnp.dot(agg, wn_ref[...], preferred_element_type=jnp.float32)
             + b_ref[...])
        o_ref[...] = jnp.maximum(o, 0.0)

    return pl.pallas_call(
        body,
        grid=(_NNB,),
        in_specs=[_nrow_spec(h.shape[1]), _part_spec(P.shape[2]),
                  _part_spec(128), _w_spec(Ws), _w_spec(Wn), _w_spec(We8),
                  _w_spec(b)],
        out_specs=_nrow_spec(dout),
        out_shape=jax.ShapeDtypeStruct((N, dout), jnp.float32),
    )(h, P, Pea, Ws, Wn, We8, b)


def _head_call(h2, P2, Pea, eps, mWs, mWn, mWe8, mb, lWs, lWn, lWe8, lb,
               aW1, ab1):
    def body(h_ref, p_ref, pe_ref, eps_ref, mws_ref, mwn_ref, mwe_ref, mb_ref,
             lws_ref, lwn_ref, lwe_ref, lb_ref, aw1_ref, ab1_ref,
             mu_ref, lv_ref, z_ref, y_ref, st_ref):
        i = pl.program_id(0)
        h = h_ref[...]
        sh = p_ref[0] + p_ref[1]
        se = (pe_ref[0] + pe_ref[1])[:, 0:8]
        deg = jnp.maximum(se[:, 4:5], 1.0)
        agg_m = (sh + jnp.dot(se, mwe_ref[...],
                              preferred_element_type=jnp.float32)) / deg
        agg_l = (sh + jnp.dot(se, lwe_ref[...],
                              preferred_element_type=jnp.float32)) / deg
        mu = (jnp.dot(h, mws_ref[...], preferred_element_type=jnp.float32)
              + jnp.dot(agg_m, mwn_ref[...], preferred_element_type=jnp.float32)
              + mb_ref[...])
        lv = (jnp.dot(h, lws_ref[...], preferred_element_type=jnp.float32)
              + jnp.dot(agg_l, lwn_ref[...], preferred_element_type=jnp.float32)
              + lb_ref[...])
        z = mu + eps_ref[...] * jnp.exp(0.5 * lv)
        mu_ref[...] = mu
        lv_ref[...] = lv
        z_ref[...] = jnp.concatenate(
            [z, jnp.zeros((_NBLK, 128 - D_OUT), jnp.float32)],
            axis=1)
        y = jnp.dot(z, aw1_ref[...], preferred_element_type=jnp.float32) + ab1_ref[...]
        y_ref[...] = y

        @pl.when(i == 0)
        def _():
            st_ref[...] = jnp.zeros_like(st_ref)

        st_ref[0:1, :] += jnp.sum(y, axis=0, keepdims=True)
        st_ref[1:2, :] += jnp.sum(y * y, axis=0, keepdims=True)

    return pl.pallas_call(
        body,
        grid=(_NNB,),
        in_specs=[_nrow_spec(D_H), _part_spec(D_H), _part_spec(128),
                  _nrow_spec(D_OUT), _w_spec(mWs), _w_spec(mWn), _w_spec(mWe8),
                  _w_spec(mb), _w_spec(lWs), _w_spec(lWn), _w_spec(lWe8),
                  _w_spec(lb), _w_spec(aW1), _w_spec(ab1)],
        out_specs=(_nrow_spec(D_OUT), _nrow_spec(D_OUT), _nrow_spec(128),
                   _nrow_spec(2 * D_OUT),
                   pl.BlockSpec((8, 2 * D_OUT), lambda i: (0, 0))),
        out_shape=(
            jax.ShapeDtypeStruct((N, D_OUT), jnp.float32),
            jax.ShapeDtypeStruct((N, D_OUT), jnp.float32),
            jax.ShapeDtypeStruct((N, 128), jnp.float32),
            jax.ShapeDtypeStruct((N, 2 * D_OUT), jnp.float32),
            jax.ShapeDtypeStruct((8, 2 * D_OUT), jnp.float32),
        ),
    )(h2, P2, Pea, eps, mWs, mWn, mWe8, mb, lWs, lWn, lWe8, lb, aW1, ab1)


def _atom_apply_call(y, st, ag, abe, aW2, ab2):
    def body(y_ref, st_ref, ag_ref, abe_ref, aw2_ref, ab2_ref, o_ref):
        y = y_ref[...]
        m = st_ref[0:1, :] / float(N)
        v = st_ref[1:2, :] / float(N) - m * m
        yh = jnp.maximum(ag_ref[...] * (y - m) / jnp.sqrt(v + 1e-5)
                         + abe_ref[...], 0.0)
        o_ref[...] = (jnp.dot(yh, aw2_ref[...], preferred_element_type=jnp.float32)
                      + ab2_ref[...])

    return pl.pallas_call(
        body,
        grid=(_NNB,),
        in_specs=[_nrow_spec(2 * D_OUT),
                  pl.BlockSpec((8, 2 * D_OUT), lambda i: (0, 0)),
                  _w_spec(ag), _w_spec(abe), _w_spec(aW2), _w_spec(ab2)],
        out_specs=_nrow_spec(N_ATOM),
        out_shape=jax.ShapeDtypeStruct((N, N_ATOM), jnp.float32),
    )(y, st, ag, abe, aW2, ab2)


_RBLK = 8000
_NEB = ES // _RBLK


def _edge_stats_call(g, W1a, W1b, b1):
    dh = W1a.shape[1]

    def body(g0_ref, g1_ref, wa_ref, wb_ref, b_ref, o_ref):
        i = pl.program_id(0)
        ga = g0_ref[...][:, 0:D_OUT]
        gb = g1_ref[...][:, 0:D_OUT]
        y = (jnp.dot(ga, wa_ref[...], preferred_element_type=jnp.float32)
             + jnp.dot(gb, wb_ref[...], preferred_element_type=jnp.float32)
             + b_ref[...])

        @pl.when(i == 0)
        def _():
            o_ref[...] = jnp.zeros_like(o_ref)

        o_ref[0:1, :] += jnp.sum(y, axis=0, keepdims=True)
        o_ref[1:2, :] += jnp.sum(y * y, axis=0, keepdims=True)

    return pl.pallas_call(
        body,
        grid=(_NEB,),
        in_specs=[
            pl.BlockSpec((_RBLK, 128), lambda i: (i, 0)),
            pl.BlockSpec((_RBLK, 128), lambda i: (i, 0)),
            pl.BlockSpec((D_OUT, dh), lambda i: (0, 0)),
            pl.BlockSpec((D_OUT, dh), lambda i: (0, 0)),
            pl.BlockSpec((1, dh), lambda i: (0, 0)),
        ],
        out_specs=pl.BlockSpec((8, dh), lambda i: (0, 0)),
        out_shape=jax.ShapeDtypeStruct((8, dh), jnp.float32),
    )(g[0], g[1], W1a, W1b, b1)


def _edge_apply_call(g, stats, W1a, W1b, b1, eg, ebe, W2, b2):
    dh = W1a.shape[1]

    def body(g0_ref, g1_ref, st_ref, wa_ref, wb_ref, b_ref, g_ref, be_ref,
             w2_ref, b2_ref, o_ref):
        ga = g0_ref[...][:, 0:D_OUT]
        gb = g1_ref[...][:, 0:D_OUT]
        y = (jnp.dot(ga, wa_ref[...], preferred_element_type=jnp.float32)
             + jnp.dot(gb, wb_ref[...], preferred_element_type=jnp.float32)
             + b_ref[...])
        m = st_ref[0:1, :] / float(ES)
        v = st_ref[1:2, :] / float(ES) - m * m
        yh = jnp.maximum(g_ref[...] * (y - m) / jnp.sqrt(v + 1e-5) + be_ref[...], 0.0)
        o_ref[...] = (jnp.dot(yh, w2_ref[...], preferred_element_type=jnp.float32)
                      + b2_ref[...])

    return pl.pallas_call(
        body,
        grid=(_NEB,),
        in_specs=[
            pl.BlockSpec((_RBLK, 128), lambda i: (i, 0)),
            pl.BlockSpec((_RBLK, 128), lambda i: (i, 0)),
            pl.BlockSpec((8, dh), lambda i: (0, 0)),
            pl.BlockSpec((D_OUT, dh), lambda i: (0, 0)),
            pl.BlockSpec((D_OUT, dh), lambda i: (0, 0)),
            pl.BlockSpec((1, dh), lambda i: (0, 0)),
            pl.BlockSpec((1, dh), lambda i: (0, 0)),
            pl.BlockSpec((1, dh), lambda i: (0, 0)),
            pl.BlockSpec((dh, 4), lambda i: (0, 0)),
            pl.BlockSpec((1, 4), lambda i: (0, 0)),
        ],
        out_specs=pl.BlockSpec((_RBLK, 4), lambda i: (i, 0)),
        out_shape=jax.ShapeDtypeStruct((ES, 4), jnp.float32),
    )(g[0], g[1], stats, W1a, W1b, b1, eg, ebe, W2, b2)


def kernel(x, edge_index, edge_attr, sampled_edge_index, eps, params):
    p = params
    f32 = jnp.float32

    x2 = x.astype(jnp.int32).reshape(N, 1)
    src = edge_index[0].astype(jnp.int32)
    dst = edge_index[1].astype(jnp.int32)

    padn = E_PAD - E
    pi = jnp.arange(padn, dtype=jnp.int32) % 128
    src2d = jnp.concatenate([src, pi]).reshape(NW * K_E, CH)
    dst2d = jnp.concatenate([dst, N + pi]).reshape(NW * K_E, CH)


    sp = jnp.arange(HALF - ES, dtype=jnp.int32) % 128
    s0 = sampled_edge_index[0].astype(jnp.int32)
    s1 = sampled_edge_index[1].astype(jnp.int32)
    sall2d = jnp.concatenate([s0, sp, s1, sp]).reshape(NW * K_S, CH)

    z128 = jnp.zeros((N_ACC, D_IN), f32)

    embp = jnp.concatenate([p['emb'], jnp.zeros((32 - N_ATOM, D_IN), f32)], axis=0)

    def we8(w):
        return jnp.concatenate([w, jnp.zeros((4, w.shape[1]), f32)], axis=0)

    def row(v):
        return v.reshape(1, -1)

    h0 = _embed_call(x2, embp)
    Pea = _seg_linear_call(_expand_ea_call(edge_attr.astype(f32)), dst2d, z128)
    P0 = _seg_gather_call(src2d, dst2d, h0, z128)
    h1 = _sage_dense_call(h0, P0, Pea, p['c1_Ws'], p['c1_Wn'], we8(p['c1_We']),
                          row(p['c1_b']))
    P1 = _seg_gather_call(src2d, dst2d, h1, z128)
    h2 = _sage_dense_call(h1, P1, Pea, p['c2_Ws'], p['c2_Wn'], we8(p['c2_We']),
                          row(p['c2_b']))
    P2 = _seg_gather_call(src2d, dst2d, h2, z128)
    mu, logvar, z, y_at, st_at = _head_call(
        h2, P2, Pea, eps,
        p['mu_Ws'], p['mu_Wn'], we8(p['mu_We']), row(p['mu_b']),
        p['lv_Ws'], p['lv_Wn'], we8(p['lv_We']), row(p['lv_b']),
        p['at_W1'], row(p['at_b1']))
    atom = _atom_apply_call(y_at, st_at, row(p['at_g']), row(p['at_be']),
                            p['at_W2'], row(p['at_b2']))

    g = _pair_gather_call(sall2d, z)
    W1a = p['el_W1'][:D_OUT]
    W1b = p['el_W1'][D_OUT:]
    stats = _edge_stats_call(g, W1a, W1b, row(p['el_b1']))
    epred = _edge_apply_call(g, stats, W1a, W1b, row(p['el_b1']),
                             row(p['el_g']), row(p['el_be']),
                             p['el_W2'], row(p['el_b2']))
    return (atom, epred, mu, logvar)


# speedup vs baseline: 6.9303x; 1.0025x over previous
import functools

import jax
import jax.numpy as jnp
from jax import lax
from jax.experimental import pallas as pl
from jax.experimental.pallas import tpu as pltpu
from jax.experimental.pallas import tpu_sc as plsc

N = 10000
E = 320000
ES = 160000
D_IN = 128
D_H = 128
D_OUT = 64
N_ATOM = 28

NC = 2
NS = 16
NW = NC * NS
CH = 128
K_E = 80
E_PAD = NW * K_E * CH
RPT_E = K_E * CH
N_ACC = 10240
NZR = N_ACC // NS
HALF = ES + 3840
K_S = 80
_MESH = plsc.VectorSubcoreMesh(core_axis_name="c", subcore_axis_name="s")


def _seg_gather_call(src2d, dst2d, table, zeros):
    d = table.shape[1]

    @functools.partial(
        pl.kernel,
        out_type=jax.ShapeDtypeStruct((NC, N_ACC, d), jnp.float32),
        mesh=_MESH,
        scratch_types=[
            pltpu.VMEM((K_E, CH), jnp.int32),
            pltpu.VMEM((K_E // 2, CH), jnp.int32),
            pltpu.VMEM((CH, d), jnp.float32),
            pltpu.VMEM((CH, d), jnp.float32),
            pltpu.SemaphoreType.DMA,
            pltpu.SemaphoreType.DMA,
            pltpu.VMEM_SHARED((N_ACC, d), jnp.float32),
        ],
    )
    def k(src_h, dst_h, tab_h, zero_h, out_h, sidx, didx, ra, rb, sa, sb, acc):
        c = lax.axis_index("c")
        s = lax.axis_index("s")
        wid = c * NS + s
        kh = K_E // 2
        pltpu.sync_copy(zero_h.at[pl.ds(s * NZR, NZR)], acc.at[pl.ds(s * NZR, NZR)])
        pltpu.sync_copy(src_h.at[pl.ds(wid * K_E, K_E)], sidx)
        plsc.subcore_barrier()

        for ph in range(2):
            pltpu.sync_copy(dst_h.at[pl.ds(wid * K_E + ph * kh, kh)], didx)
            pltpu.async_copy(tab_h.at[sidx.at[ph * kh]], ra, sa)

            @pl.loop(0, kh // 2)
            def _(t):
                j = ph * kh + 2 * t
                pltpu.async_copy(tab_h.at[sidx.at[j + 1]], rb, sb)
                pltpu.make_async_copy(tab_h.at[sidx.at[j]], ra, sa).wait()
                pltpu.sync_copy(ra, acc.at[didx.at[2 * t]], add=True)

                @pl.when(2 * t + 2 < kh)
                def _():
                    pltpu.async_copy(tab_h.at[sidx.at[j + 2]], ra, sa)

                pltpu.make_async_copy(tab_h.at[sidx.at[j + 1]], rb, sb).wait()
                pltpu.sync_copy(rb, acc.at[didx.at[2 * t + 1]], add=True)

        plsc.subcore_barrier()
        pltpu.sync_copy(acc.at[pl.ds(s * NZR, NZR)],
                        out_h.at[c].at[pl.ds(s * NZR, NZR)])

    return k(src2d, dst2d, table, zeros)


def _seg_linear_call(vals, dst2d, zeros):
    @functools.partial(
        pl.kernel,
        out_type=jax.ShapeDtypeStruct((NC, N_ACC, 128), jnp.float32),
        mesh=_MESH,
        scratch_types=[
            pltpu.VMEM((K_E, CH), jnp.int32),
            pltpu.VMEM((CH, 128), jnp.float32),
            pltpu.VMEM((CH, 128), jnp.float32),
            pltpu.SemaphoreType.DMA,
            pltpu.SemaphoreType.DMA,
            pltpu.VMEM_SHARED((N_ACC, 128), jnp.float32),
        ],
    )
    def k(val_h, dst_h, zero_h, out_h, didx, ra, rb, sa, sb, acc):
        c = lax.axis_index("c")
        s = lax.axis_index("s")
        wid = c * NS + s
        base = wid * RPT_E
        pltpu.sync_copy(zero_h.at[pl.ds(s * NZR, NZR)], acc.at[pl.ds(s * NZR, NZR)])
        pltpu.sync_copy(dst_h.at[pl.ds(wid * K_E, K_E)], didx)
        plsc.subcore_barrier()
        pltpu.async_copy(val_h.at[pl.ds(base, CH)], ra, sa)

        @pl.loop(0, K_E // 2)
        def _(t):
            j = 2 * t
            pltpu.async_copy(val_h.at[pl.ds(base + (j + 1) * CH, CH)], rb, sb)
            pltpu.make_async_copy(val_h.at[pl.ds(base, CH)], ra, sa).wait()
            pltpu.sync_copy(ra, acc.at[didx.at[j]], add=True)

            @pl.when(j + 2 < K_E)
            def _():
                pltpu.async_copy(val_h.at[pl.ds(base + (j + 2) * CH, CH)], ra, sa)

            pltpu.make_async_copy(val_h.at[pl.ds(base, CH)], rb, sb).wait()
            pltpu.sync_copy(rb, acc.at[didx.at[j + 1]], add=True)

        plsc.subcore_barrier()
        pltpu.sync_copy(acc.at[pl.ds(s * NZR, NZR)],
                        out_h.at[c].at[pl.ds(s * NZR, NZR)])

    return k(vals, dst2d, zeros)


_EABLK = 8192


def _expand_ea_call(ea):
    def body(a_ref, o_ref):
        o_ref[...] = jnp.concatenate(
            [a_ref[...],
             jnp.ones((_EABLK, 1), jnp.float32),
             jnp.zeros((_EABLK, 123), jnp.float32)], axis=1)

    return pl.pallas_call(
        body,
        grid=(E_PAD // _EABLK,),
        in_specs=[pl.BlockSpec((_EABLK, 4), lambda i: (i, 0))],
        out_specs=pl.BlockSpec((_EABLK, 128), lambda i: (i, 0)),
        out_shape=jax.ShapeDtypeStruct((E_PAD, 128), jnp.float32),
    )(ea)


def _pair_gather_call(idx2d, zw):
    @functools.partial(
        pl.kernel,
        out_type=(jax.ShapeDtypeStruct((HALF, 128), jnp.float32),
                  jax.ShapeDtypeStruct((HALF, 128), jnp.float32)),
        mesh=_MESH,
        scratch_types=[
            pltpu.VMEM((K_S, CH), jnp.int32),
            pltpu.VMEM((CH, 128), jnp.float32),
            pltpu.VMEM((CH, 128), jnp.float32),
            pltpu.SemaphoreType.DMA,
            pltpu.SemaphoreType.DMA,
        ],
    )
    def k(idx_h, tab_h, out0_h, out1_h, gidx, ra, rb, sa, sb):
        c = lax.axis_index("c")
        s = lax.axis_index("s")
        wid = c * NS + s
        half = wid // 16
        rbase = (wid % 16) * (K_S * CH)
        pltpu.sync_copy(idx_h.at[pl.ds(wid * K_S, K_S)], gidx)
        pltpu.async_copy(tab_h.at[gidx.at[0]], ra, sa)

        def wr(buf, j):
            @pl.when(half == 0)
            def _():
                pltpu.sync_copy(buf, out0_h.at[pl.ds(rbase + j * CH, CH)])

            @pl.when(half == 1)
            def _():
                pltpu.sync_copy(buf, out1_h.at[pl.ds(rbase + j * CH, CH)])

        @pl.loop(0, K_S // 2)
        def _(t):
            j = 2 * t
            pltpu.async_copy(tab_h.at[gidx.at[j + 1]], rb, sb)
            pltpu.make_async_copy(tab_h.at[gidx.at[j]], ra, sa).wait()
            wr(ra, j)

            @pl.when(j + 2 < K_S)
            def _():
                pltpu.async_copy(tab_h.at[gidx.at[j + 2]], ra, sa)

            pltpu.make_async_copy(tab_h.at[gidx.at[j + 1]], rb, sb).wait()
            wr(rb, j + 1)

    return k(idx2d, zw)


def _embed_call(x2, embp):
    def body(x_ref, e_ref, o_ref):
        oh = (x_ref[...] == lax.broadcasted_iota(jnp.int32, (N, 32), 1))
        o_ref[...] = jnp.dot(oh.astype(jnp.float32), e_ref[...],
                             preferred_element_type=jnp.float32)

    return pl.pallas_call(
        body, out_shape=jax.ShapeDtypeStruct((N, D_IN), jnp.float32),
    )(x2, embp)


_NBLK = 5000
_NNB = N // _NBLK


def _nrow_spec(d):
    return pl.BlockSpec((_NBLK, d), lambda i: (i, 0))


def _part_spec(d):
    return pl.BlockSpec((2, _NBLK, d), lambda i: (0, i, 0))


def _w_spec(a):
    return pl.BlockSpec(a.shape, lambda i: tuple(0 for _ in a.shape))


def _sage_dense_call(h, P, Pea, Ws, Wn, We8, b):
    dout = Wn.shape[1]

    def body(h_ref, p_ref, pe_ref, ws_ref, wn_ref, we_ref, b_ref, o_ref):
        sh = p_ref[0] + p_ref[1]
        se = (pe_ref[0] + pe_ref[1])[:, 0:8]
        deg = jnp.maximum(se[:, 4:5], 1.0)
        agg = (sh + jnp.dot(se, we_ref[...],
                            preferred_element_type=jnp.float32)) / deg
        o = (jnp.dot(h_ref[...], ws_ref[...], preferred_element_type=jnp.float32)
             + jnp.dot(agg, wn_ref[...], preferred_element_type=jnp.float32)
             + b_ref[...])
        o_ref[...] = jnp.maximum(o, 0.0)

    return pl.pallas_call(
        body,
        grid=(_NNB,),
        in_specs=[_nrow_spec(h.shape[1]), _part_spec(P.shape[2]),
                  _part_spec(128), _w_spec(Ws), _w_spec(Wn), _w_spec(We8),
                  _w_spec(b)],
        out_specs=_nrow_spec(dout),
        out_shape=jax.ShapeDtypeStruct((N, dout), jnp.float32),
    )(h, P, Pea, Ws, Wn, We8, b)


def _head_call(h2, P2, Pea, eps, mWs, mWn, mWe8, mb, lWs, lWn, lWe8, lb,
               aW1, ab1):
    def body(h_ref, p_ref, pe_ref, eps_ref, mws_ref, mwn_ref, mwe_ref, mb_ref,
             lws_ref, lwn_ref, lwe_ref, lb_ref, aw1_ref, ab1_ref,
             mu_ref, lv_ref, z_ref, y_ref, st_ref):
        i = pl.program_id(0)
        h = h_ref[...]
        sh = p_ref[0] + p_ref[1]
        se = (pe_ref[0] + pe_ref[1])[:, 0:8]
        deg = jnp.maximum(se[:, 4:5], 1.0)
        agg_m = (sh + jnp.dot(se, mwe_ref[...],
                              preferred_element_type=jnp.float32)) / deg
        agg_l = (sh + jnp.dot(se, lwe_ref[...],
                              preferred_element_type=jnp.float32)) / deg
        mu = (jnp.dot(h, mws_ref[...], preferred_element_type=jnp.float32)
              + jnp.dot(agg_m, mwn_ref[...], preferred_element_type=jnp.float32)
              + mb_ref[...])
        lv = (jnp.dot(h, lws_ref[...], preferred_element_type=jnp.float32)
              + jnp.dot(agg_l, lwn_ref[...], preferred_element_type=jnp.float32)
              + lb_ref[...])
        z = mu + eps_ref[...] * jnp.exp(0.5 * lv)
        mu_ref[...] = mu
        lv_ref[...] = lv
        z_ref[...] = jnp.concatenate(
            [z, jnp.zeros((_NBLK, 128 - D_OUT), jnp.float32)],
            axis=1)
        y = jnp.dot(z, aw1_ref[...], preferred_element_type=jnp.float32) + ab1_ref[...]
        y_ref[...] = y

        @pl.when(i == 0)
        def _():
            st_ref[...] = jnp.zeros_like(st_ref)

        st_ref[0:1, :] += jnp.sum(y, axis=0, keepdims=True)
        st_ref[1:2, :] += jnp.sum(y * y, axis=0, keepdims=True)

    return pl.pallas_call(
        body,
        grid=(_NNB,),
        in_specs=[_nrow_spec(D_H), _part_spec(D_H), _part_spec(128),
                  _nrow_spec(D_OUT), _w_spec(mWs), _w_spec(mWn), _w_spec(mWe8),
                  _w_spec(mb), _w_spec(lWs), _w_spec(lWn), _w_spec(lWe8),
                  _w_spec(lb), _w_spec(aW1), _w_spec(ab1)],
        out_specs=(_nrow_spec(D_OUT), _nrow_spec(D_OUT), _nrow_spec(128),
                   _nrow_spec(2 * D_OUT),
                   pl.BlockSpec((8, 2 * D_OUT), lambda i: (0, 0))),
        out_shape=(
            jax.ShapeDtypeStruct((N, D_OUT), jnp.float32),
            jax.ShapeDtypeStruct((N, D_OUT), jnp.float32),
            jax.ShapeDtypeStruct((N, 128), jnp.float32),
            jax.ShapeDtypeStruct((N, 2 * D_OUT), jnp.float32),
            jax.ShapeDtypeStruct((8, 2 * D_OUT), jnp.float32),
        ),
    )(h2, P2, Pea, eps, mWs, mWn, mWe8, mb, lWs, lWn, lWe8, lb, aW1, ab1)


def _atom_apply_call(y, st, ag, abe, aW2, ab2):
    def body(y_ref, st_ref, ag_ref, abe_ref, aw2_ref, ab2_ref, o_ref):
        y = y_ref[...]
        m = st_ref[0:1, :] / float(N)
        v = st_ref[1:2, :] / float(N) - m * m
        yh = jnp.maximum(ag_ref[...] * (y - m) / jnp.sqrt(v + 1e-5)
                         + abe_ref[...], 0.0)
        o_ref[...] = (jnp.dot(yh, aw2_ref[...], preferred_element_type=jnp.float32)
                      + ab2_ref[...])

    return pl.pallas_call(
        body,
        grid=(_NNB,),
        in_specs=[_nrow_spec(2 * D_OUT),
                  pl.BlockSpec((8, 2 * D_OUT), lambda i: (0, 0)),
                  _w_spec(ag), _w_spec(abe), _w_spec(aW2), _w_spec(ab2)],
        out_specs=_nrow_spec(N_ATOM),
        out_shape=jax.ShapeDtypeStruct((N, N_ATOM), jnp.float32),
    )(y, st, ag, abe, aW2, ab2)


_RBLK = 10000
_NEB = ES // _RBLK


def _edge_stats_call(g, W1a, W1b, b1):
    dh = W1a.shape[1]

    def body(g0_ref, g1_ref, wa_ref, wb_ref, b_ref, o_ref):
        i = pl.program_id(0)
        ga = g0_ref[...][:, 0:D_OUT]
        gb = g1_ref[...][:, 0:D_OUT]
        y = (jnp.dot(ga, wa_ref[...], preferred_element_type=jnp.float32)
             + jnp.dot(gb, wb_ref[...], preferred_element_type=jnp.float32)
             + b_ref[...])

        @pl.when(i == 0)
        def _():
            o_ref[...] = jnp.zeros_like(o_ref)

        o_ref[0:1, :] += jnp.sum(y, axis=0, keepdims=True)
        o_ref[1:2, :] += jnp.sum(y * y, axis=0, keepdims=True)

    return pl.pallas_call(
        body,
        grid=(_NEB,),
        in_specs=[
            pl.BlockSpec((_RBLK, 128), lambda i: (i, 0)),
            pl.BlockSpec((_RBLK, 128), lambda i: (i, 0)),
            pl.BlockSpec((D_OUT, dh), lambda i: (0, 0)),
            pl.BlockSpec((D_OUT, dh), lambda i: (0, 0)),
            pl.BlockSpec((1, dh), lambda i: (0, 0)),
        ],
        out_specs=pl.BlockSpec((8, dh), lambda i: (0, 0)),
        out_shape=jax.ShapeDtypeStruct((8, dh), jnp.float32),
    )(g[0], g[1], W1a, W1b, b1)


def _edge_apply_call(g, stats, W1a, W1b, b1, eg, ebe, W2, b2):
    dh = W1a.shape[1]

    def body(g0_ref, g1_ref, st_ref, wa_ref, wb_ref, b_ref, g_ref, be_ref,
             w2_ref, b2_ref, o_ref):
        ga = g0_ref[...][:, 0:D_OUT]
        gb = g1_ref[...][:, 0:D_OUT]
        y = (jnp.dot(ga, wa_ref[...], preferred_element_type=jnp.float32)
             + jnp.dot(gb, wb_ref[...], preferred_element_type=jnp.float32)
             + b_ref[...])
        m = st_ref[0:1, :] / float(ES)
        v = st_ref[1:2, :] / float(ES) - m * m
        yh = jnp.maximum(g_ref[...] * (y - m) / jnp.sqrt(v + 1e-5) + be_ref[...], 0.0)
        o_ref[...] = (jnp.dot(yh, w2_ref[...], preferred_element_type=jnp.float32)
                      + b2_ref[...])

    return pl.pallas_call(
        body,
        grid=(_NEB,),
        in_specs=[
            pl.BlockSpec((_RBLK, 128), lambda i: (i, 0)),
            pl.BlockSpec((_RBLK, 128), lambda i: (i, 0)),
            pl.BlockSpec((8, dh), lambda i: (0, 0)),
            pl.BlockSpec((D_OUT, dh), lambda i: (0, 0)),
            pl.BlockSpec((D_OUT, dh), lambda i: (0, 0)),
            pl.BlockSpec((1, dh), lambda i: (0, 0)),
            pl.BlockSpec((1, dh), lambda i: (0, 0)),
            pl.BlockSpec((1, dh), lambda i: (0, 0)),
            pl.BlockSpec((dh, 4), lambda i: (0, 0)),
            pl.BlockSpec((1, 4), lambda i: (0, 0)),
        ],
        out_specs=pl.BlockSpec((_RBLK, 4), lambda i: (i, 0)),
        out_shape=jax.ShapeDtypeStruct((ES, 4), jnp.float32),
    )(g[0], g[1], stats, W1a, W1b, b1, eg, ebe, W2, b2)


def kernel(x, edge_index, edge_attr, sampled_edge_index, eps, params):
    p = params
    f32 = jnp.float32

    x2 = x.astype(jnp.int32).reshape(N, 1)
    src = edge_index[0].astype(jnp.int32)
    dst = edge_index[1].astype(jnp.int32)

    padn = E_PAD - E
    pi = jnp.arange(padn, dtype=jnp.int32) % 128
    src2d = jnp.concatenate([src, pi]).reshape(NW * K_E, CH)
    dst2d = jnp.concatenate([dst, N + pi]).reshape(NW * K_E, CH)


    sp = jnp.arange(HALF - ES, dtype=jnp.int32) % 128
    s0 = sampled_edge_index[0].astype(jnp.int32)
    s1 = sampled_edge_index[1].astype(jnp.int32)
    sall2d = jnp.concatenate([s0, sp, s1, sp]).reshape(NW * K_S, CH)

    z128 = jnp.zeros((N_ACC, D_IN), f32)

    embp = jnp.concatenate([p['emb'], jnp.zeros((32 - N_ATOM, D_IN), f32)], axis=0)

    def we8(w):
        return jnp.concatenate([w, jnp.zeros((4, w.shape[1]), f32)], axis=0)

    def row(v):
        return v.reshape(1, -1)

    h0 = _embed_call(x2, embp)
    Pea = _seg_linear_call(_expand_ea_call(edge_attr.astype(f32)), dst2d, z128)
    P0 = _seg_gather_call(src2d, dst2d, h0, z128)
    h1 = _sage_dense_call(h0, P0, Pea, p['c1_Ws'], p['c1_Wn'], we8(p['c1_We']),
                          row(p['c1_b']))
    P1 = _seg_gather_call(src2d, dst2d, h1, z128)
    h2 = _sage_dense_call(h1, P1, Pea, p['c2_Ws'], p['c2_Wn'], we8(p['c2_We']),
                          row(p['c2_b']))
    P2 = _seg_gather_call(src2d, dst2d, h2, z128)
    mu, logvar, z, y_at, st_at = _head_call(
        h2, P2, Pea, eps,
        p['mu_Ws'], p['mu_Wn'], we8(p['mu_We']), row(p['mu_b']),
        p['lv_Ws'], p['lv_Wn'], we8(p['lv_We']), row(p['lv_b']),
        p['at_W1'], row(p['at_b1']))
    atom = _atom_apply_call(y_at, st_at, row(p['at_g']), row(p['at_be']),
                            p['at_W2'], row(p['at_b2']))

    g = _pair_gather_call(sall2d, z)
    W1a = p['el_W1'][:D_OUT]
    W1b = p['el_W1'][D_OUT:]
    stats = _edge_stats_call(g, W1a, W1b, row(p['el_b1']))
    epred = _edge_apply_call(g, stats, W1a, W1b, row(p['el_b1']),
                             row(p['el_g']), row(p['el_be']),
                             p['el_W2'], row(p['el_b2']))
    return (atom, epred, mu, logvar)
